# baseline (speedup 1.0000x reference)
import functools

import jax
import jax.numpy as jnp
from jax import lax
from jax.experimental import pallas as pl
from jax.experimental.pallas import tpu as pltpu
from jax.experimental.pallas import tpu_sc as plsc

D = 2048
H = 16
HD = 128
E = 64
TOPK = 8
M = 1024
SCALE = 0.08838834764831845
S = 2048
EPS = 1e-05

TILE = 128
PN = S * TOPK + E * TILE
NTILES = PN // TILE
NA = S * TOPK

SB = 256
NSB = S // SB


def _norm1_body(x_ref, w_ref, o_ref):
    x = x_ref[...]
    o_ref[...] = x * lax.rsqrt(jnp.mean(x * x, axis=-1, keepdims=True) + EPS) * w_ref[...]


def _rmsnorm(x2d, w):
    return pl.pallas_call(
        _norm1_body,
        grid=(NSB,),
        in_specs=[pl.BlockSpec((SB, D), lambda i: (i, 0)),
                  pl.BlockSpec((1, D), lambda i: (0, 0))],
        out_specs=pl.BlockSpec((SB, D), lambda i: (i, 0)),
        out_shape=jax.ShapeDtypeStruct((S, D), jnp.float32),
    )(x2d, w.reshape(1, D))


def _qkv_body(xn_ref, w_ref, o_ref):
    o_ref[...] = lax.dot_general(xn_ref[...], w_ref[...],
                                 (((1,), (1,)), ((), ())),
                                 preferred_element_type=jnp.float32)


def _qkv_proj(xn, wqkv):
    return pl.pallas_call(
        _qkv_body,
        grid=(6, NSB),
        in_specs=[pl.BlockSpec((SB, D), lambda j, i: (i, 0)),
                  pl.BlockSpec((1024, D), lambda j, i: (j, 0))],
        out_specs=pl.BlockSpec((SB, 1024), lambda j, i: (i, j)),
        out_shape=jax.ShapeDtypeStruct((S, 3 * D), jnp.float32),
    )(xn, wqkv)


def _rope_body(q_ref, w_ref, o_ref):
    q = q_ref[...]
    blk = q.shape[0]
    qn = q * lax.rsqrt(jnp.mean(q * q, axis=(1, 2), keepdims=True) + EPS) * w_ref[...]
    i = pl.program_id(0)
    pos = (i * blk + lax.broadcasted_iota(jnp.int32, (blk, 1, 1), 0)).astype(jnp.float32)
    lanes = lax.broadcasted_iota(jnp.int32, (1, 1, HD), 2)
    f = (lanes % 64).astype(jnp.float32)
    inv_freq = jnp.exp(f * (-jnp.log(10000.0) / 64.0))
    ang = pos * inv_freq
    c, s = jnp.cos(ang), jnp.sin(ang)
    a = qn * c
    b = qn * s
    b_sw = pltpu.roll(b, 64, 2)
    o_ref[...] = a + jnp.where(lanes < 64, -b_sw, b_sw)


def _rope(q3, w3):
    return pl.pallas_call(
        _rope_body,
        grid=(16,),
        in_specs=[pl.BlockSpec((128, H, HD), lambda i: (i, 0, 0)),
                  pl.BlockSpec((1, H, HD), lambda i: (0, 0, 0))],
        out_specs=pl.BlockSpec((128, H, HD), lambda i: (i, 0, 0)),
        out_shape=jax.ShapeDtypeStruct((S, H, HD), jnp.float32),
    )(q3, w3)


def _attn_body(q_ref, k_ref, v_ref, o_ref):
    i = pl.program_id(1)
    q = q_ref[0]
    k = k_ref[0]
    v = v_ref[0]
    s = lax.dot_general(q, k, (((1,), (1,)), ((), ())),
                        preferred_element_type=jnp.float32) * SCALE
    rows = i * SB + lax.broadcasted_iota(jnp.int32, (SB, S), 0)
    cols = lax.broadcasted_iota(jnp.int32, (SB, S), 1)
    s = jnp.where(cols <= rows, s, -jnp.inf)
    m = jnp.max(s, axis=1, keepdims=True)
    p = jnp.exp(s - m)
    z = jnp.sum(p, axis=1, keepdims=True)
    o_ref[0] = lax.dot_general(p, v, (((1,), (0,)), ((), ())),
                               preferred_element_type=jnp.float32) / z


def _attention(qh, kh, vh):
    return pl.pallas_call(
        _attn_body,
        grid=(H, NSB),
        in_specs=[pl.BlockSpec((1, SB, HD), lambda h, i: (h, i, 0)),
                  pl.BlockSpec((1, S, HD), lambda h, i: (h, 0, 0)),
                  pl.BlockSpec((1, S, HD), lambda h, i: (h, 0, 0))],
        out_specs=pl.BlockSpec((1, SB, HD), lambda h, i: (h, i, 0)),
        out_shape=jax.ShapeDtypeStruct((H, S, HD), jnp.float32),
    )(qh, kh, vh)


def _oproj_body(c_ref, w_ref, x_ref, o_ref):
    o_ref[...] = lax.dot_general(c_ref[...], w_ref[...],
                                 (((1,), (1,)), ((), ())),
                                 preferred_element_type=jnp.float32) + x_ref[...]


def _oproj_res(ctx, o_w, x2d):
    return pl.pallas_call(
        _oproj_body,
        grid=(4, NSB),
        in_specs=[pl.BlockSpec((SB, D), lambda j, i: (i, 0)),
                  pl.BlockSpec((512, D), lambda j, i: (j, 0)),
                  pl.BlockSpec((SB, 512), lambda j, i: (i, j))],
        out_specs=pl.BlockSpec((SB, 512), lambda j, i: (i, j)),
        out_shape=jax.ShapeDtypeStruct((S, D), jnp.float32),
    )(ctx, o_w, x2d)


def _router_body(x2_ref, ln2_ref, gw_ref, x3_ref, wp_ref, ip_ref):
    x2 = x2_ref[...]
    x3 = x2 * lax.rsqrt(jnp.mean(x2 * x2, axis=-1, keepdims=True) + EPS) * ln2_ref[...]
    x3_ref[...] = x3
    logits = lax.dot_general(x3, gw_ref[...], (((1,), (1,)), ((), ())),
                             preferred_element_type=jnp.float32)
    mx = jnp.max(logits, axis=1, keepdims=True)
    ex = jnp.exp(logits - mx)
    probs = ex / jnp.sum(ex, axis=1, keepdims=True)
    cols = lax.broadcasted_iota(jnp.int32, (SB, E), 1)
    lane = lax.broadcasted_iota(jnp.int32, (SB, 128), 1)
    accw = jnp.zeros((SB, 128), jnp.float32)
    acci = jnp.zeros((SB, 128), jnp.int32)
    for j in range(TOPK):
        m = jnp.max(probs, axis=1, keepdims=True)
        idx = jnp.min(jnp.where(probs == m, cols, E), axis=1, keepdims=True)
        accw = jnp.where(lane == j, m, accw)
        acci = jnp.where(lane == j, idx, acci)
        probs = jnp.where(cols == idx, -1.0, probs)
    wp_ref[...] = accw
    ip_ref[...] = acci


def _router(x2, ln2_w, gate_w):
    return pl.pallas_call(
        _router_body,
        grid=(NSB,),
        in_specs=[pl.BlockSpec((SB, D), lambda i: (i, 0)),
                  pl.BlockSpec((1, D), lambda i: (0, 0)),
                  pl.BlockSpec((E, D), lambda i: (0, 0))],
        out_specs=[pl.BlockSpec((SB, D), lambda i: (i, 0)),
                   pl.BlockSpec((SB, 128), lambda i: (i, 0)),
                   pl.BlockSpec((SB, 128), lambda i: (i, 0))],
        out_shape=[jax.ShapeDtypeStruct((S, D), jnp.float32),
                   jax.ShapeDtypeStruct((S, 128), jnp.float32),
                   jax.ShapeDtypeStruct((S, 128), jnp.int32)],
    )(x2, ln2_w.reshape(1, D), gate_w)


def _meta_body(ip_ref, rank_ref, cnt_ref, carry):
    i = pl.program_id(0)

    @pl.when(i == 0)
    def _():
        carry[...] = jnp.zeros_like(carry)

    erange = lax.broadcasted_iota(jnp.int32, (1, E), 1)
    ohs = []
    for k in range(TOPK):
        ek = ip_ref[:, k:k + 1]
        ohs.append((ek == erange).astype(jnp.float32))
    rowsum = sum(ohs)
    r = lax.broadcasted_iota(jnp.int32, (SB, SB), 0)
    c = lax.broadcasted_iota(jnp.int32, (SB, SB), 1)
    ltri = (c < r).astype(jnp.float32)
    rowpre = lax.dot_general(ltri, rowsum, (((1,), (0,)), ((), ())),
                             preferred_element_type=jnp.float32)
    acc = rowpre + carry[0:1, 0:E]
    lane = lax.broadcasted_iota(jnp.int32, (SB, 128), 1)
    rk = jnp.zeros((SB, 128), jnp.float32)
    for k in range(TOPK):
        rkk = jnp.sum(ohs[k] * acc, axis=1, keepdims=True)
        rk = jnp.where(lane == k, rkk, rk)
        acc = acc + ohs[k]
    rank_ref[...] = rk
    carry[0:1, 0:E] = carry[0:1, 0:E] + jnp.sum(rowsum, axis=0, keepdims=True)
    cnt_ref[...] = jnp.broadcast_to(carry[0:1, :], (8, 128))


def _meta(ipad):
    return pl.pallas_call(
        _meta_body,
        grid=(NSB,),
        in_specs=[pl.BlockSpec((SB, 128), lambda i: (i, 0))],
        out_specs=[pl.BlockSpec((SB, 128), lambda i: (i, 0)),
                   pl.BlockSpec((8, 128), lambda i: (0, 0))],
        out_shape=[jax.ShapeDtypeStruct((S, 128), jnp.float32),
                   jax.ShapeDtypeStruct((8, 128), jnp.float32)],
        scratch_shapes=[pltpu.VMEM((8, 128), jnp.float32)],
    )(ipad)


def _poff_body(cnt_ref, poff_ref, te_ref):
    c = cnt_ref[0:1, 0:E]
    nt = jnp.floor((c + (TILE - 1.0)) * (1.0 / TILE))
    r = lax.broadcasted_iota(jnp.int32, (E, E), 0)
    cc = lax.broadcasted_iota(jnp.int32, (E, E), 1)
    utri = (r < cc).astype(jnp.float32)
    ts = lax.dot_general(nt, utri, (((1,), (0,)), ((), ())),
                         preferred_element_type=jnp.float32)
    poff_ref[...] = jnp.zeros((8, 128), jnp.float32)
    poff_ref[0:1, 0:E] = ts * TILE
    ts_col = jnp.transpose(ts, (1, 0))
    ti = lax.broadcasted_iota(jnp.int32, (1, NTILES), 1).astype(jnp.float32)
    te = jnp.sum((ti >= ts_col).astype(jnp.float32), axis=0, keepdims=True) - 1.0
    te = jnp.clip(te, 0.0, E - 1.0)
    te_ref[...] = jnp.broadcast_to(te, (8, NTILES))
    te_ref[1:2, 0:1] = jnp.sum(nt, axis=1, keepdims=True)


def _poff(cnt):
    return pl.pallas_call(
        _poff_body,
        grid=(1,),
        in_specs=[pl.BlockSpec((8, 128), lambda i: (0, 0))],
        out_specs=[pl.BlockSpec((8, 128), lambda i: (0, 0)),
                   pl.BlockSpec((8, NTILES), lambda i: (0, 0))],
        out_shape=[jax.ShapeDtypeStruct((8, 128), jnp.float32),
                   jax.ShapeDtypeStruct((8, NTILES), jnp.float32)],
    )(cnt)


def _dest_body(ip_ref, rank_ref, poff_ref, dest_ref):
    erange = lax.broadcasted_iota(jnp.int32, (1, E), 1)
    lane = lax.broadcasted_iota(jnp.int32, (SB, 128), 1)
    poff = poff_ref[0:1, 0:E]
    add = jnp.zeros((SB, 128), jnp.float32)
    for k in range(TOPK):
        ek = ip_ref[:, k:k + 1]
        oh = (ek == erange).astype(jnp.float32)
        pe = jnp.sum(oh * poff, axis=1, keepdims=True)
        add = jnp.where(lane == k, pe, add)
    dest_ref[...] = rank_ref[...] + add


def _dest(ipad, rankpad, poff):
    return pl.pallas_call(
        _dest_body,
        grid=(NSB,),
        in_specs=[pl.BlockSpec((SB, 128), lambda i: (i, 0)),
                  pl.BlockSpec((SB, 128), lambda i: (i, 0)),
                  pl.BlockSpec((8, 128), lambda i: (0, 0))],
        out_specs=pl.BlockSpec((SB, 128), lambda i: (i, 0)),
        out_shape=jax.ShapeDtypeStruct((S, 128), jnp.float32),
    )(ipad, rankpad, poff)


def _wcast_body(gup_ref, dwn_ref, gub_ref, dwb_ref):
    gub_ref[...] = gup_ref[...].astype(jnp.bfloat16)
    dwb_ref[...] = dwn_ref[...].astype(jnp.bfloat16)


def _wcast(gate_up_w, down_w):
    return pl.pallas_call(
        _wcast_body,
        grid=(E, 2),
        in_specs=[pl.BlockSpec((1, M, D), lambda e, j: (e, j, 0)),
                  pl.BlockSpec((1, M, M), lambda e, j: (e, j, 0))],
        out_specs=[pl.BlockSpec((1, M, D), lambda e, j: (e, j, 0)),
                   pl.BlockSpec((1, M, M), lambda e, j: (e, j, 0))],
        out_shape=[jax.ShapeDtypeStruct((E, 2 * M, D), jnp.bfloat16),
                   jax.ShapeDtypeStruct((E, D, M), jnp.bfloat16)],
    )(gate_up_w, down_w)


def _ffn_body(ntu_ref, te_ref, xg_ref, gup_ref, dwn_ref, o_ref):
    @pl.when(pl.program_id(0) < ntu_ref[0])
    def _():
        x = xg_ref[...].astype(jnp.bfloat16)
        gu = lax.dot_general(x, gup_ref[0], (((1,), (1,)), ((), ())),
                             preferred_element_type=jnp.float32)
        g = gu[:, :M]
        u = gu[:, M:]
        act = (g * (1.0 / (1.0 + jnp.exp(-g))) * u).astype(jnp.bfloat16)
        o_ref[...] = lax.dot_general(act, dwn_ref[0], (((1,), (1,)), ((), ())),
                                     preferred_element_type=jnp.float32)


def _ffn(ntu, te, xg, gate_up_w, down_w):
    grid_spec = pltpu.PrefetchScalarGridSpec(
        num_scalar_prefetch=2,
        grid=(NTILES,),
        in_specs=[
            pl.BlockSpec((TILE, D), lambda i, ntu, te: (i, 0)),
            pl.BlockSpec((1, 2 * M, D), lambda i, ntu, te: (te[i], 0, 0)),
            pl.BlockSpec((1, D, M), lambda i, ntu, te: (te[i], 0, 0)),
        ],
        out_specs=pl.BlockSpec((TILE, D), lambda i, ntu, te: (i, 0)),
    )
    return pl.pallas_call(
        _ffn_body,
        grid_spec=grid_spec,
        out_shape=jax.ShapeDtypeStruct((PN, D), jnp.float32),
    )(ntu, te, xg, gate_up_w, down_w)


def _comb_body(hg_ref, wp_ref, x2_ref, o_ref):
    k = pl.program_id(1)
    lane = lax.broadcasted_iota(jnp.int32, (SB, 128), 1)
    wk = jnp.sum(jnp.where(lane == k, wp_ref[...], 0.0), axis=1, keepdims=True)
    contrib = hg_ref[...] * wk

    @pl.when(k == 0)
    def _():
        o_ref[...] = x2_ref[...] + contrib

    @pl.when(k > 0)
    def _():
        o_ref[...] = o_ref[...] + contrib


def _combine(hg, wpad, x2):
    return pl.pallas_call(
        _comb_body,
        grid=(NSB, TOPK),
        in_specs=[pl.BlockSpec((SB, D), lambda i, k: (k * NSB + i, 0)),
                  pl.BlockSpec((SB, 128), lambda i, k: (i, 0)),
                  pl.BlockSpec((SB, D), lambda i, k: (i, 0))],
        out_specs=pl.BlockSpec((SB, D), lambda i, k: (i, 0)),
        out_shape=jax.ShapeDtypeStruct((S, D), jnp.float32),
    )(hg, wpad, x2)


_NW = 32


def _sc_scatter_rows(src, idx3):
    mesh = plsc.VectorSubcoreMesh(core_axis_name="c", subcore_axis_name="s")
    rows_per_w = NA // _NW
    nchunk = rows_per_w // 128

    @functools.partial(
        pl.kernel, mesh=mesh,
        out_type=jax.ShapeDtypeStruct((PN, 128), jnp.int32),
        scratch_types=[pltpu.VMEM((nchunk, 128), jnp.int32),
                       pltpu.VMEM((128, 128), jnp.int32),
                       pltpu.SemaphoreType.DMA],
    )
    def k(src_hbm, idx_hbm, out_hbm, idx_v, src_v, sem):
        wid = lax.axis_index("s") * 2 + lax.axis_index("c")
        base = wid * rows_per_w
        pltpu.sync_copy(idx_hbm.at[wid], idx_v)
        for j in range(nchunk):
            pltpu.sync_copy(src_hbm.at[pl.ds(base + j * 128, 128)], src_v)
            pltpu.async_copy(src_v, out_hbm.at[idx_v.at[j]], sem).wait()

    return k(src, idx3)


def _sc_gather_rows(table, idx, nrows, clamp_max):
    mesh = plsc.VectorSubcoreMesh(core_axis_name="c", subcore_axis_name="s")
    rows_per_w = nrows // _NW
    chunk = 24 if rows_per_w % 24 == 0 else 16
    nchunk = rows_per_w // chunk
    dt = table.dtype
    ncol = table.shape[1]

    @functools.partial(
        pl.kernel, mesh=mesh,
        out_type=jax.ShapeDtypeStruct((nrows, ncol), dt),
        scratch_types=[pltpu.VMEM((rows_per_w,), jnp.int32),
                       pltpu.VMEM((chunk, ncol), dt),
                       pltpu.VMEM((chunk, ncol), dt),
                       pltpu.SemaphoreType.DMA,
                       pltpu.SemaphoreType.DMA,
                       pltpu.SemaphoreType.DMA,
                       pltpu.SemaphoreType.DMA],
    )
    def k(tab_hbm, idx_hbm, out_hbm, idx_v, buf0, buf1, gs0, gs1, ws0, ws1):
        wid = lax.axis_index("s") * 2 + lax.axis_index("c")
        base = wid * rows_per_w
        bufs = (buf0, buf1)
        gsems = (gs0, gs1)
        wsems = (ws0, ws1)

        pltpu.sync_copy(idx_hbm.at[pl.ds(base, rows_per_w)], idx_v)
        for h in range(rows_per_w // 16):
            v = idx_v[pl.ds(h * 16, 16)]
            idx_v[pl.ds(h * 16, 16)] = jnp.minimum(jnp.maximum(v, 0), clamp_max)

        def gstart(c, b):
            pltpu.async_copy(tab_hbm.at[idx_v.at[pl.ds(c * chunk, chunk)]],
                             bufs[b], gsems[b])

        def gwait(b):
            pltpu.make_async_copy(tab_hbm.at[pl.ds(0, chunk)], bufs[b],
                                  gsems[b]).wait()

        def wstart(c, b):
            pltpu.async_copy(bufs[b], out_hbm.at[pl.ds(base + c * chunk, chunk)],
                             wsems[b])

        def wwait(b):
            pltpu.make_async_copy(tab_hbm.at[pl.ds(0, chunk)], bufs[b],
                                  wsems[b]).wait()

        gstart(0, 0)

        def step(c, carry):
            even = c % 2 == 0

            @pl.when(c >= 1)
            def _():
                @pl.when(even)
                def _():
                    wwait(1)

                @pl.when(jnp.logical_not(even))
                def _():
                    wwait(0)

            @pl.when(c + 1 < nchunk)
            def _():
                @pl.when(even)
                def _():
                    gstart(c + 1, 1)

                @pl.when(jnp.logical_not(even))
                def _():
                    gstart(c + 1, 0)

            @pl.when(even)
            def _():
                gwait(0)
                wstart(c, 0)

            @pl.when(jnp.logical_not(even))
            def _():
                gwait(1)
                wstart(c, 1)

            return carry

        lax.fori_loop(0, nchunk, step, 0)
        wwait((nchunk - 1) % 2)

    return k(table, idx)


def kernel(x, ln1_w, q_w, k_w, v_w, qn_w, kn_w, o_w, ln2_w, gate_w, gate_up_w, down_w):
    x2d = x.reshape(S, D)

    xn = _rmsnorm(x2d, ln1_w)
    wqkv = jnp.concatenate([q_w, k_w, v_w], axis=0)
    qkv = _qkv_proj(xn, wqkv)
    q3 = qkv[:, :D].reshape(S, H, HD)
    k3 = qkv[:, D:2 * D].reshape(S, H, HD)
    v3 = qkv[:, 2 * D:].reshape(S, H, HD)

    qr = _rope(q3, qn_w.reshape(1, H, HD))
    kr = _rope(k3, kn_w.reshape(1, H, HD))

    qh = qr.transpose(1, 0, 2)
    kh = kr.transpose(1, 0, 2)
    vh = v3.transpose(1, 0, 2)
    ctxh = _attention(qh, kh, vh)
    ctx = ctxh.transpose(1, 0, 2).reshape(S, D)

    x2 = _oproj_res(ctx, o_w, x2d)
    x3, wpad, ipad = _router(x2, ln2_w, gate_w)

    rankpad, cnt = _meta(ipad)
    poff, tef = _poff(cnt)
    destpad = _dest(ipad, rankpad, poff)

    te = tef[0, :].astype(jnp.int32)
    dest_i = destpad[:, :TOPK].astype(jnp.int32)
    scat_idx = dest_i.reshape(_NW, NA // _NW // 128, 128)
    tok = jnp.repeat(jnp.arange(S, dtype=jnp.int32), TOPK)
    scat_src = jnp.broadcast_to(tok[:, None], (NA, 128))

    glist_rows = _sc_scatter_rows(scat_src, scat_idx)
    glist = glist_rows[:, 0]

    ntu = tef[1, 0:1].astype(jnp.int32)
    xg = _sc_gather_rows(x3, glist, PN, S - 1)
    gub, dwb = _wcast(gate_up_w, down_w)
    h_sorted = _ffn(ntu, te, xg, gub, dwb)

    cidx = dest_i.T.reshape(NA)
    hg = _sc_gather_rows(h_sorted, cidx, NA, PN - 1)

    out = _combine(hg, wpad, x2)
    return out.reshape(1, S, D)


# speedup vs baseline: 1.1849x; 1.1849x over previous
import functools

import jax
import jax.numpy as jnp
from jax import lax
from jax.experimental import pallas as pl
from jax.experimental.pallas import tpu as pltpu
from jax.experimental.pallas import tpu_sc as plsc

D = 2048
H = 16
HD = 128
E = 64
TOPK = 8
M = 1024
SCALE = 0.08838834764831845
S = 2048
EPS = 1e-05

TILE = 128
PN = S * TOPK + E * TILE
NTILES = PN // TILE
NA = S * TOPK

SB = 256
NSB = S // SB


def _norm1_body(x_ref, w_ref, o_ref):
    x = x_ref[...]
    o_ref[...] = x * lax.rsqrt(jnp.mean(x * x, axis=-1, keepdims=True) + EPS) * w_ref[...]


def _rmsnorm(x2d, w):
    return pl.pallas_call(
        _norm1_body,
        grid=(NSB,),
        in_specs=[pl.BlockSpec((SB, D), lambda i: (i, 0)),
                  pl.BlockSpec((1, D), lambda i: (0, 0))],
        out_specs=pl.BlockSpec((SB, D), lambda i: (i, 0)),
        out_shape=jax.ShapeDtypeStruct((S, D), jnp.float32),
    )(x2d, w.reshape(1, D))


def _qkv_body(xn_ref, w_ref, o_ref):
    o_ref[...] = lax.dot_general(xn_ref[...], w_ref[...],
                                 (((1,), (1,)), ((), ())),
                                 preferred_element_type=jnp.float32)


def _qkv_proj(xn, wqkv):
    return pl.pallas_call(
        _qkv_body,
        grid=(6, NSB),
        in_specs=[pl.BlockSpec((SB, D), lambda j, i: (i, 0)),
                  pl.BlockSpec((1024, D), lambda j, i: (j, 0))],
        out_specs=pl.BlockSpec((SB, 1024), lambda j, i: (i, j)),
        out_shape=jax.ShapeDtypeStruct((S, 3 * D), jnp.float32),
    )(xn, wqkv)


def _rope_body(q_ref, w_ref, o_ref):
    q = q_ref[...]
    blk = q.shape[0]
    qn = q * lax.rsqrt(jnp.mean(q * q, axis=(1, 2), keepdims=True) + EPS) * w_ref[...]
    i = pl.program_id(0)
    pos = (i * blk + lax.broadcasted_iota(jnp.int32, (blk, 1, 1), 0)).astype(jnp.float32)
    lanes = lax.broadcasted_iota(jnp.int32, (1, 1, HD), 2)
    f = (lanes % 64).astype(jnp.float32)
    inv_freq = jnp.exp(f * (-jnp.log(10000.0) / 64.0))
    ang = pos * inv_freq
    c, s = jnp.cos(ang), jnp.sin(ang)
    a = qn * c
    b = qn * s
    b_sw = pltpu.roll(b, 64, 2)
    o_ref[...] = a + jnp.where(lanes < 64, -b_sw, b_sw)


def _rope(q3, w3):
    return pl.pallas_call(
        _rope_body,
        grid=(16,),
        in_specs=[pl.BlockSpec((128, H, HD), lambda i: (i, 0, 0)),
                  pl.BlockSpec((1, H, HD), lambda i: (0, 0, 0))],
        out_specs=pl.BlockSpec((128, H, HD), lambda i: (i, 0, 0)),
        out_shape=jax.ShapeDtypeStruct((S, H, HD), jnp.float32),
    )(q3, w3)


def _attn_body(q_ref, k_ref, v_ref, o_ref):
    i = pl.program_id(1)
    q = q_ref[0]
    k = k_ref[0]
    v = v_ref[0]
    s = lax.dot_general(q, k, (((1,), (1,)), ((), ())),
                        preferred_element_type=jnp.float32) * SCALE
    rows = i * SB + lax.broadcasted_iota(jnp.int32, (SB, S), 0)
    cols = lax.broadcasted_iota(jnp.int32, (SB, S), 1)
    s = jnp.where(cols <= rows, s, -jnp.inf)
    m = jnp.max(s, axis=1, keepdims=True)
    p = jnp.exp(s - m)
    z = jnp.sum(p, axis=1, keepdims=True)
    o_ref[0] = lax.dot_general(p, v, (((1,), (0,)), ((), ())),
                               preferred_element_type=jnp.float32) / z


def _attention(qh, kh, vh):
    return pl.pallas_call(
        _attn_body,
        grid=(H, NSB),
        in_specs=[pl.BlockSpec((1, SB, HD), lambda h, i: (h, i, 0)),
                  pl.BlockSpec((1, S, HD), lambda h, i: (h, 0, 0)),
                  pl.BlockSpec((1, S, HD), lambda h, i: (h, 0, 0))],
        out_specs=pl.BlockSpec((1, SB, HD), lambda h, i: (h, i, 0)),
        out_shape=jax.ShapeDtypeStruct((H, S, HD), jnp.float32),
    )(qh, kh, vh)


def _oproj_body(c_ref, w_ref, x_ref, o_ref):
    o_ref[...] = lax.dot_general(c_ref[...], w_ref[...],
                                 (((1,), (1,)), ((), ())),
                                 preferred_element_type=jnp.float32) + x_ref[...]


def _oproj_res(ctx, o_w, x2d):
    return pl.pallas_call(
        _oproj_body,
        grid=(4, NSB),
        in_specs=[pl.BlockSpec((SB, D), lambda j, i: (i, 0)),
                  pl.BlockSpec((512, D), lambda j, i: (j, 0)),
                  pl.BlockSpec((SB, 512), lambda j, i: (i, j))],
        out_specs=pl.BlockSpec((SB, 512), lambda j, i: (i, j)),
        out_shape=jax.ShapeDtypeStruct((S, D), jnp.float32),
    )(ctx, o_w, x2d)


def _router_body(x2_ref, ln2_ref, gw_ref, x3_ref, wp_ref, ip_ref):
    x2 = x2_ref[...]
    x3 = x2 * lax.rsqrt(jnp.mean(x2 * x2, axis=-1, keepdims=True) + EPS) * ln2_ref[...]
    x3_ref[...] = x3
    logits = lax.dot_general(x3, gw_ref[...], (((1,), (1,)), ((), ())),
                             preferred_element_type=jnp.float32)
    mx = jnp.max(logits, axis=1, keepdims=True)
    ex = jnp.exp(logits - mx)
    probs = ex / jnp.sum(ex, axis=1, keepdims=True)
    cols = lax.broadcasted_iota(jnp.int32, (SB, E), 1)
    lane = lax.broadcasted_iota(jnp.int32, (SB, 128), 1)
    accw = jnp.zeros((SB, 128), jnp.float32)
    acci = jnp.zeros((SB, 128), jnp.int32)
    for j in range(TOPK):
        m = jnp.max(probs, axis=1, keepdims=True)
        idx = jnp.min(jnp.where(probs == m, cols, E), axis=1, keepdims=True)
        accw = jnp.where(lane == j, m, accw)
        acci = jnp.where(lane == j, idx, acci)
        probs = jnp.where(cols == idx, -1.0, probs)
    wp_ref[...] = accw
    ip_ref[...] = acci


def _router(x2, ln2_w, gate_w):
    return pl.pallas_call(
        _router_body,
        grid=(NSB,),
        in_specs=[pl.BlockSpec((SB, D), lambda i: (i, 0)),
                  pl.BlockSpec((1, D), lambda i: (0, 0)),
                  pl.BlockSpec((E, D), lambda i: (0, 0))],
        out_specs=[pl.BlockSpec((SB, D), lambda i: (i, 0)),
                   pl.BlockSpec((SB, 128), lambda i: (i, 0)),
                   pl.BlockSpec((SB, 128), lambda i: (i, 0))],
        out_shape=[jax.ShapeDtypeStruct((S, D), jnp.float32),
                   jax.ShapeDtypeStruct((S, 128), jnp.float32),
                   jax.ShapeDtypeStruct((S, 128), jnp.int32)],
    )(x2, ln2_w.reshape(1, D), gate_w)


def _meta_body(ip_ref, rank_ref, cnt_ref, carry):
    i = pl.program_id(0)

    @pl.when(i == 0)
    def _():
        carry[...] = jnp.zeros_like(carry)

    erange = lax.broadcasted_iota(jnp.int32, (1, E), 1)
    ohs = []
    for k in range(TOPK):
        ek = ip_ref[:, k:k + 1]
        ohs.append((ek == erange).astype(jnp.float32))
    rowsum = sum(ohs)
    r = lax.broadcasted_iota(jnp.int32, (SB, SB), 0)
    c = lax.broadcasted_iota(jnp.int32, (SB, SB), 1)
    ltri = (c < r).astype(jnp.float32)
    rowpre = lax.dot_general(ltri, rowsum, (((1,), (0,)), ((), ())),
                             preferred_element_type=jnp.float32)
    acc = rowpre + carry[0:1, 0:E]
    lane = lax.broadcasted_iota(jnp.int32, (SB, 128), 1)
    rk = jnp.zeros((SB, 128), jnp.float32)
    for k in range(TOPK):
        rkk = jnp.sum(ohs[k] * acc, axis=1, keepdims=True)
        rk = jnp.where(lane == k, rkk, rk)
        acc = acc + ohs[k]
    rank_ref[...] = rk
    carry[0:1, 0:E] = carry[0:1, 0:E] + jnp.sum(rowsum, axis=0, keepdims=True)
    cnt_ref[...] = jnp.broadcast_to(carry[0:1, :], (8, 128))


def _meta(ipad):
    return pl.pallas_call(
        _meta_body,
        grid=(NSB,),
        in_specs=[pl.BlockSpec((SB, 128), lambda i: (i, 0))],
        out_specs=[pl.BlockSpec((SB, 128), lambda i: (i, 0)),
                   pl.BlockSpec((8, 128), lambda i: (0, 0))],
        out_shape=[jax.ShapeDtypeStruct((S, 128), jnp.float32),
                   jax.ShapeDtypeStruct((8, 128), jnp.float32)],
        scratch_shapes=[pltpu.VMEM((8, 128), jnp.float32)],
    )(ipad)


def _poff_body(cnt_ref, poff_ref, te_ref):
    c = cnt_ref[0:1, 0:E]
    nt = jnp.floor((c + (TILE - 1.0)) * (1.0 / TILE))
    r = lax.broadcasted_iota(jnp.int32, (E, E), 0)
    cc = lax.broadcasted_iota(jnp.int32, (E, E), 1)
    utri = (r < cc).astype(jnp.float32)
    ts = lax.dot_general(nt, utri, (((1,), (0,)), ((), ())),
                         preferred_element_type=jnp.float32)
    poff_ref[...] = jnp.zeros((8, 128), jnp.float32)
    poff_ref[0:1, 0:E] = ts * TILE
    ts_col = jnp.transpose(ts, (1, 0))
    ti = lax.broadcasted_iota(jnp.int32, (1, NTILES), 1).astype(jnp.float32)
    te = jnp.sum((ti >= ts_col).astype(jnp.float32), axis=0, keepdims=True) - 1.0
    te = jnp.clip(te, 0.0, E - 1.0)
    te_ref[...] = jnp.broadcast_to(te, (8, NTILES))
    te_ref[1:2, 0:1] = jnp.sum(nt, axis=1, keepdims=True)


def _poff(cnt):
    return pl.pallas_call(
        _poff_body,
        grid=(1,),
        in_specs=[pl.BlockSpec((8, 128), lambda i: (0, 0))],
        out_specs=[pl.BlockSpec((8, 128), lambda i: (0, 0)),
                   pl.BlockSpec((8, NTILES), lambda i: (0, 0))],
        out_shape=[jax.ShapeDtypeStruct((8, 128), jnp.float32),
                   jax.ShapeDtypeStruct((8, NTILES), jnp.float32)],
    )(cnt)


def _dest_body(ip_ref, rank_ref, poff_ref, dest_ref):
    erange = lax.broadcasted_iota(jnp.int32, (1, E), 1)
    lane = lax.broadcasted_iota(jnp.int32, (SB, 128), 1)
    poff = poff_ref[0:1, 0:E]
    add = jnp.zeros((SB, 128), jnp.float32)
    for k in range(TOPK):
        ek = ip_ref[:, k:k + 1]
        oh = (ek == erange).astype(jnp.float32)
        pe = jnp.sum(oh * poff, axis=1, keepdims=True)
        add = jnp.where(lane == k, pe, add)
    dest_ref[...] = rank_ref[...] + add


def _dest(ipad, rankpad, poff):
    return pl.pallas_call(
        _dest_body,
        grid=(NSB,),
        in_specs=[pl.BlockSpec((SB, 128), lambda i: (i, 0)),
                  pl.BlockSpec((SB, 128), lambda i: (i, 0)),
                  pl.BlockSpec((8, 128), lambda i: (0, 0))],
        out_specs=pl.BlockSpec((SB, 128), lambda i: (i, 0)),
        out_shape=jax.ShapeDtypeStruct((S, 128), jnp.float32),
    )(ipad, rankpad, poff)


def _wcast_body(gup_ref, dwn_ref, gub_ref, dwb_ref):
    gub_ref[...] = gup_ref[...].astype(jnp.bfloat16)
    dwb_ref[...] = dwn_ref[...].astype(jnp.bfloat16)


def _wcast(gate_up_w, down_w):
    return pl.pallas_call(
        _wcast_body,
        grid=(E, 2),
        in_specs=[pl.BlockSpec((1, M, D), lambda e, j: (e, j, 0)),
                  pl.BlockSpec((1, M, M), lambda e, j: (e, j, 0))],
        out_specs=[pl.BlockSpec((1, M, D), lambda e, j: (e, j, 0)),
                   pl.BlockSpec((1, M, M), lambda e, j: (e, j, 0))],
        out_shape=[jax.ShapeDtypeStruct((E, 2 * M, D), jnp.bfloat16),
                   jax.ShapeDtypeStruct((E, D, M), jnp.bfloat16)],
    )(gate_up_w, down_w)


def _ffn_body(ntu_ref, te_ref, xg_ref, gup_ref, dwn_ref, o_ref):
    @pl.when(pl.program_id(0) < ntu_ref[0])
    def _():
        x = xg_ref[...]
        gu = lax.dot_general(x, gup_ref[0], (((1,), (1,)), ((), ())),
                             preferred_element_type=jnp.float32)
        g = gu[:, :M]
        u = gu[:, M:]
        act = g * (1.0 / (1.0 + jnp.exp(-g))) * u
        o_ref[...] = lax.dot_general(act, dwn_ref[0], (((1,), (1,)), ((), ())),
                                     preferred_element_type=jnp.float32)


def _ffn(ntu, te, xg, gate_up_w, down_w):
    grid_spec = pltpu.PrefetchScalarGridSpec(
        num_scalar_prefetch=2,
        grid=(NTILES,),
        in_specs=[
            pl.BlockSpec((TILE, D), lambda i, ntu, te: (i, 0)),
            pl.BlockSpec((1, 2 * M, D), lambda i, ntu, te: (te[i], 0, 0)),
            pl.BlockSpec((1, D, M), lambda i, ntu, te: (te[i], 0, 0)),
        ],
        out_specs=pl.BlockSpec((TILE, D), lambda i, ntu, te: (i, 0)),
    )
    return pl.pallas_call(
        _ffn_body,
        grid_spec=grid_spec,
        out_shape=jax.ShapeDtypeStruct((PN, D), jnp.float32),
    )(ntu, te, xg, gate_up_w, down_w)


def _comb_body(hg_ref, wp_ref, x2_ref, o_ref):
    k = pl.program_id(1)
    lane = lax.broadcasted_iota(jnp.int32, (SB, 128), 1)
    wk = jnp.sum(jnp.where(lane == k, wp_ref[...], 0.0), axis=1, keepdims=True)
    contrib = hg_ref[...] * wk

    @pl.when(k == 0)
    def _():
        o_ref[...] = x2_ref[...] + contrib

    @pl.when(k > 0)
    def _():
        o_ref[...] = o_ref[...] + contrib


def _combine(hg, wpad, x2):
    return pl.pallas_call(
        _comb_body,
        grid=(NSB, TOPK),
        in_specs=[pl.BlockSpec((SB, D), lambda i, k: (k * NSB + i, 0)),
                  pl.BlockSpec((SB, 128), lambda i, k: (i, 0)),
                  pl.BlockSpec((SB, D), lambda i, k: (i, 0))],
        out_specs=pl.BlockSpec((SB, D), lambda i, k: (i, 0)),
        out_shape=jax.ShapeDtypeStruct((S, D), jnp.float32),
    )(hg, wpad, x2)


_NW = 32


def _sc_scatter_rows(src, idx3):
    mesh = plsc.VectorSubcoreMesh(core_axis_name="c", subcore_axis_name="s")
    rows_per_w = NA // _NW
    nchunk = rows_per_w // 128

    @functools.partial(
        pl.kernel, mesh=mesh,
        out_type=jax.ShapeDtypeStruct((PN, 128), jnp.int32),
        scratch_types=[pltpu.VMEM((nchunk, 128), jnp.int32),
                       pltpu.VMEM((128, 128), jnp.int32),
                       pltpu.SemaphoreType.DMA],
    )
    def k(src_hbm, idx_hbm, out_hbm, idx_v, src_v, sem):
        wid = lax.axis_index("s") * 2 + lax.axis_index("c")
        base = wid * rows_per_w
        pltpu.sync_copy(idx_hbm.at[wid], idx_v)
        for j in range(nchunk):
            pltpu.sync_copy(src_hbm.at[pl.ds(base + j * 128, 128)], src_v)
            pltpu.async_copy(src_v, out_hbm.at[idx_v.at[j]], sem).wait()

    return k(src, idx3)


def _sc_gather_rows(table, idx, nrows, clamp_max):
    mesh = plsc.VectorSubcoreMesh(core_axis_name="c", subcore_axis_name="s")
    rows_per_w = nrows // _NW
    chunk = 24 if rows_per_w % 24 == 0 else 16
    nchunk = rows_per_w // chunk
    dt = table.dtype
    ncol = table.shape[1]

    @functools.partial(
        pl.kernel, mesh=mesh,
        out_type=jax.ShapeDtypeStruct((nrows, ncol), dt),
        scratch_types=[pltpu.VMEM((rows_per_w,), jnp.int32),
                       pltpu.VMEM((chunk, ncol), dt),
                       pltpu.VMEM((chunk, ncol), dt),
                       pltpu.SemaphoreType.DMA,
                       pltpu.SemaphoreType.DMA,
                       pltpu.SemaphoreType.DMA,
                       pltpu.SemaphoreType.DMA],
    )
    def k(tab_hbm, idx_hbm, out_hbm, idx_v, buf0, buf1, gs0, gs1, ws0, ws1):
        wid = lax.axis_index("s") * 2 + lax.axis_index("c")
        base = wid * rows_per_w
        bufs = (buf0, buf1)
        gsems = (gs0, gs1)
        wsems = (ws0, ws1)

        pltpu.sync_copy(idx_hbm.at[pl.ds(base, rows_per_w)], idx_v)
        for h in range(rows_per_w // 16):
            v = idx_v[pl.ds(h * 16, 16)]
            idx_v[pl.ds(h * 16, 16)] = jnp.minimum(jnp.maximum(v, 0), clamp_max)

        def gstart(c, b):
            pltpu.async_copy(tab_hbm.at[idx_v.at[pl.ds(c * chunk, chunk)]],
                             bufs[b], gsems[b])

        def gwait(b):
            pltpu.make_async_copy(tab_hbm.at[pl.ds(0, chunk)], bufs[b],
                                  gsems[b]).wait()

        def wstart(c, b):
            pltpu.async_copy(bufs[b], out_hbm.at[pl.ds(base + c * chunk, chunk)],
                             wsems[b])

        def wwait(b):
            pltpu.make_async_copy(tab_hbm.at[pl.ds(0, chunk)], bufs[b],
                                  wsems[b]).wait()

        gstart(0, 0)

        def step(c, carry):
            even = c % 2 == 0

            @pl.when(c >= 1)
            def _():
                @pl.when(even)
                def _():
                    wwait(1)

                @pl.when(jnp.logical_not(even))
                def _():
                    wwait(0)

            @pl.when(c + 1 < nchunk)
            def _():
                @pl.when(even)
                def _():
                    gstart(c + 1, 1)

                @pl.when(jnp.logical_not(even))
                def _():
                    gstart(c + 1, 0)

            @pl.when(even)
            def _():
                gwait(0)
                wstart(c, 0)

            @pl.when(jnp.logical_not(even))
            def _():
                gwait(1)
                wstart(c, 1)

            return carry

        lax.fori_loop(0, nchunk, step, 0)
        wwait((nchunk - 1) % 2)

    return k(table, idx)


def kernel(x, ln1_w, q_w, k_w, v_w, qn_w, kn_w, o_w, ln2_w, gate_w, gate_up_w, down_w):
    x2d = x.reshape(S, D)

    xn = _rmsnorm(x2d, ln1_w)
    wqkv = jnp.concatenate([q_w, k_w, v_w], axis=0)
    qkv = _qkv_proj(xn, wqkv)
    q3 = qkv[:, :D].reshape(S, H, HD)
    k3 = qkv[:, D:2 * D].reshape(S, H, HD)
    v3 = qkv[:, 2 * D:].reshape(S, H, HD)

    qr = _rope(q3, qn_w.reshape(1, H, HD))
    kr = _rope(k3, kn_w.reshape(1, H, HD))

    qh = qr.transpose(1, 0, 2)
    kh = kr.transpose(1, 0, 2)
    vh = v3.transpose(1, 0, 2)
    ctxh = _attention(qh, kh, vh)
    ctx = ctxh.transpose(1, 0, 2).reshape(S, D)

    x2 = _oproj_res(ctx, o_w, x2d)
    x3, wpad, ipad = _router(x2, ln2_w, gate_w)

    rankpad, cnt = _meta(ipad)
    poff, tef = _poff(cnt)
    destpad = _dest(ipad, rankpad, poff)

    te = tef[0, :].astype(jnp.int32)
    dest_i = destpad[:, :TOPK].astype(jnp.int32)
    scat_idx = dest_i.reshape(_NW, NA // _NW // 128, 128)
    tok = jnp.repeat(jnp.arange(S, dtype=jnp.int32), TOPK)
    scat_src = jnp.broadcast_to(tok[:, None], (NA, 128))

    glist_rows = _sc_scatter_rows(scat_src, scat_idx)
    glist = glist_rows[:, 0]

    ntu = tef[1, 0:1].astype(jnp.int32)
    xg = _sc_gather_rows(x3, glist, PN, S - 1)
    h_sorted = _ffn(ntu, te, xg, gate_up_w, down_w)

    cidx = dest_i.T.reshape(NA)
    hg = _sc_gather_rows(h_sorted, cidx, NA, PN - 1)

    out = _combine(hg, wpad, x2)
    return out.reshape(1, S, D)


# speedup vs baseline: 1.5068x; 1.2717x over previous
import functools

import jax
import jax.numpy as jnp
from jax import lax
from jax.experimental import pallas as pl
from jax.experimental.pallas import tpu as pltpu
from jax.experimental.pallas import tpu_sc as plsc

D = 2048
H = 16
HD = 128
E = 64
TOPK = 8
M = 1024
SCALE = 0.08838834764831845
S = 2048
EPS = 1e-05

TILE = 128
PN = S * TOPK + E * TILE
NTILES = PN // TILE
NA = S * TOPK

SB = 256
NSB = S // SB


def _norm1_body(x_ref, w_ref, o_ref):
    x = x_ref[...]
    o_ref[...] = x * lax.rsqrt(jnp.mean(x * x, axis=-1, keepdims=True) + EPS) * w_ref[...]


def _rmsnorm(x2d, w):
    return pl.pallas_call(
        _norm1_body,
        grid=(NSB,),
        in_specs=[pl.BlockSpec((SB, D), lambda i: (i, 0)),
                  pl.BlockSpec((1, D), lambda i: (0, 0))],
        out_specs=pl.BlockSpec((SB, D), lambda i: (i, 0)),
        out_shape=jax.ShapeDtypeStruct((S, D), jnp.float32),
    )(x2d, w.reshape(1, D))


def _qkv_body(xn_ref, w_ref, o_ref):
    o_ref[...] = lax.dot_general(xn_ref[...], w_ref[...],
                                 (((1,), (1,)), ((), ())),
                                 preferred_element_type=jnp.float32)


def _qkv_proj(xn, wqkv):
    return pl.pallas_call(
        _qkv_body,
        grid=(6, NSB),
        in_specs=[pl.BlockSpec((SB, D), lambda j, i: (i, 0)),
                  pl.BlockSpec((1024, D), lambda j, i: (j, 0))],
        out_specs=pl.BlockSpec((SB, 1024), lambda j, i: (i, j)),
        out_shape=jax.ShapeDtypeStruct((S, 3 * D), jnp.float32),
    )(xn, wqkv)


def _rope_body(q_ref, w_ref, o_ref):
    q = q_ref[...]
    blk = q.shape[0]
    qn = q * lax.rsqrt(jnp.mean(q * q, axis=(1, 2), keepdims=True) + EPS) * w_ref[...]
    i = pl.program_id(0)
    pos = (i * blk + lax.broadcasted_iota(jnp.int32, (blk, 1, 1), 0)).astype(jnp.float32)
    lanes = lax.broadcasted_iota(jnp.int32, (1, 1, HD), 2)
    f = (lanes % 64).astype(jnp.float32)
    inv_freq = jnp.exp(f * (-jnp.log(10000.0) / 64.0))
    ang = pos * inv_freq
    c, s = jnp.cos(ang), jnp.sin(ang)
    a = qn * c
    b = qn * s
    b_sw = pltpu.roll(b, 64, 2)
    o_ref[...] = a + jnp.where(lanes < 64, -b_sw, b_sw)


def _rope(q3, w3):
    return pl.pallas_call(
        _rope_body,
        grid=(16,),
        in_specs=[pl.BlockSpec((128, H, HD), lambda i: (i, 0, 0)),
                  pl.BlockSpec((1, H, HD), lambda i: (0, 0, 0))],
        out_specs=pl.BlockSpec((128, H, HD), lambda i: (i, 0, 0)),
        out_shape=jax.ShapeDtypeStruct((S, H, HD), jnp.float32),
    )(q3, w3)


def _attn_body(q_ref, k_ref, v_ref, o_ref):
    i = pl.program_id(1)
    q = q_ref[0]
    k = k_ref[0]
    v = v_ref[0]
    s = lax.dot_general(q, k, (((1,), (1,)), ((), ())),
                        preferred_element_type=jnp.float32) * SCALE
    rows = i * SB + lax.broadcasted_iota(jnp.int32, (SB, S), 0)
    cols = lax.broadcasted_iota(jnp.int32, (SB, S), 1)
    s = jnp.where(cols <= rows, s, -jnp.inf)
    m = jnp.max(s, axis=1, keepdims=True)
    p = jnp.exp(s - m)
    z = jnp.sum(p, axis=1, keepdims=True)
    o_ref[0] = lax.dot_general(p, v, (((1,), (0,)), ((), ())),
                               preferred_element_type=jnp.float32) / z


def _attention(qh, kh, vh):
    return pl.pallas_call(
        _attn_body,
        grid=(H, NSB),
        in_specs=[pl.BlockSpec((1, SB, HD), lambda h, i: (h, i, 0)),
                  pl.BlockSpec((1, S, HD), lambda h, i: (h, 0, 0)),
                  pl.BlockSpec((1, S, HD), lambda h, i: (h, 0, 0))],
        out_specs=pl.BlockSpec((1, SB, HD), lambda h, i: (h, i, 0)),
        out_shape=jax.ShapeDtypeStruct((H, S, HD), jnp.float32),
    )(qh, kh, vh)


def _oproj_body(c_ref, w_ref, x_ref, o_ref):
    o_ref[...] = lax.dot_general(c_ref[...], w_ref[...],
                                 (((1,), (1,)), ((), ())),
                                 preferred_element_type=jnp.float32) + x_ref[...]


def _oproj_res(ctx, o_w, x2d):
    return pl.pallas_call(
        _oproj_body,
        grid=(4, NSB),
        in_specs=[pl.BlockSpec((SB, D), lambda j, i: (i, 0)),
                  pl.BlockSpec((512, D), lambda j, i: (j, 0)),
                  pl.BlockSpec((SB, 512), lambda j, i: (i, j))],
        out_specs=pl.BlockSpec((SB, 512), lambda j, i: (i, j)),
        out_shape=jax.ShapeDtypeStruct((S, D), jnp.float32),
    )(ctx, o_w, x2d)


def _router_body(x2_ref, ln2_ref, gw_ref, x3_ref, wp_ref, ip_ref):
    x2 = x2_ref[...]
    x3 = x2 * lax.rsqrt(jnp.mean(x2 * x2, axis=-1, keepdims=True) + EPS) * ln2_ref[...]
    x3_ref[...] = x3
    logits = lax.dot_general(x3, gw_ref[...], (((1,), (1,)), ((), ())),
                             preferred_element_type=jnp.float32)
    mx = jnp.max(logits, axis=1, keepdims=True)
    ex = jnp.exp(logits - mx)
    probs = ex / jnp.sum(ex, axis=1, keepdims=True)
    cols = lax.broadcasted_iota(jnp.int32, (SB, E), 1)
    lane = lax.broadcasted_iota(jnp.int32, (SB, 128), 1)
    accw = jnp.zeros((SB, 128), jnp.float32)
    acci = jnp.zeros((SB, 128), jnp.int32)
    for j in range(TOPK):
        m = jnp.max(probs, axis=1, keepdims=True)
        idx = jnp.min(jnp.where(probs == m, cols, E), axis=1, keepdims=True)
        accw = jnp.where(lane == j, m, accw)
        acci = jnp.where(lane == j, idx, acci)
        probs = jnp.where(cols == idx, -1.0, probs)
    wp_ref[...] = accw
    ip_ref[...] = acci


def _router(x2, ln2_w, gate_w):
    return pl.pallas_call(
        _router_body,
        grid=(NSB,),
        in_specs=[pl.BlockSpec((SB, D), lambda i: (i, 0)),
                  pl.BlockSpec((1, D), lambda i: (0, 0)),
                  pl.BlockSpec((E, D), lambda i: (0, 0))],
        out_specs=[pl.BlockSpec((SB, D), lambda i: (i, 0)),
                   pl.BlockSpec((SB, 128), lambda i: (i, 0)),
                   pl.BlockSpec((SB, 128), lambda i: (i, 0))],
        out_shape=[jax.ShapeDtypeStruct((S, D), jnp.float32),
                   jax.ShapeDtypeStruct((S, 128), jnp.float32),
                   jax.ShapeDtypeStruct((S, 128), jnp.int32)],
    )(x2, ln2_w.reshape(1, D), gate_w)


def _meta_body(ip_ref, rank_ref, cnt_ref, carry):
    i = pl.program_id(0)

    @pl.when(i == 0)
    def _():
        carry[...] = jnp.zeros_like(carry)

    erange = lax.broadcasted_iota(jnp.int32, (1, E), 1)
    ohs = []
    for k in range(TOPK):
        ek = ip_ref[:, k:k + 1]
        ohs.append((ek == erange).astype(jnp.float32))
    rowsum = sum(ohs)
    r = lax.broadcasted_iota(jnp.int32, (SB, SB), 0)
    c = lax.broadcasted_iota(jnp.int32, (SB, SB), 1)
    ltri = (c < r).astype(jnp.float32)
    rowpre = lax.dot_general(ltri, rowsum, (((1,), (0,)), ((), ())),
                             preferred_element_type=jnp.float32)
    acc = rowpre + carry[0:1, 0:E]
    lane = lax.broadcasted_iota(jnp.int32, (SB, 128), 1)
    rk = jnp.zeros((SB, 128), jnp.float32)
    for k in range(TOPK):
        rkk = jnp.sum(ohs[k] * acc, axis=1, keepdims=True)
        rk = jnp.where(lane == k, rkk, rk)
        acc = acc + ohs[k]
    rank_ref[...] = rk
    carry[0:1, 0:E] = carry[0:1, 0:E] + jnp.sum(rowsum, axis=0, keepdims=True)
    cnt_ref[...] = jnp.broadcast_to(carry[0:1, :], (8, 128))


def _meta(ipad):
    return pl.pallas_call(
        _meta_body,
        grid=(NSB,),
        in_specs=[pl.BlockSpec((SB, 128), lambda i: (i, 0))],
        out_specs=[pl.BlockSpec((SB, 128), lambda i: (i, 0)),
                   pl.BlockSpec((8, 128), lambda i: (0, 0))],
        out_shape=[jax.ShapeDtypeStruct((S, 128), jnp.float32),
                   jax.ShapeDtypeStruct((8, 128), jnp.float32)],
        scratch_shapes=[pltpu.VMEM((8, 128), jnp.float32)],
    )(ipad)


def _poff_body(cnt_ref, poff_ref, te_ref):
    c = cnt_ref[0:1, 0:E]
    nt = jnp.floor((c + (TILE - 1.0)) * (1.0 / TILE))
    r = lax.broadcasted_iota(jnp.int32, (E, E), 0)
    cc = lax.broadcasted_iota(jnp.int32, (E, E), 1)
    utri = (r < cc).astype(jnp.float32)
    ts = lax.dot_general(nt, utri, (((1,), (0,)), ((), ())),
                         preferred_element_type=jnp.float32)
    poff_ref[...] = jnp.zeros((8, 128), jnp.float32)
    poff_ref[0:1, 0:E] = ts * TILE
    ts_col = jnp.transpose(ts, (1, 0))
    ti = lax.broadcasted_iota(jnp.int32, (1, NTILES), 1).astype(jnp.float32)
    te = jnp.sum((ti >= ts_col).astype(jnp.float32), axis=0, keepdims=True) - 1.0
    te = jnp.clip(te, 0.0, E - 1.0)
    te_ref[...] = jnp.broadcast_to(te, (8, NTILES))
    te_ref[1:2, 0:1] = jnp.sum(nt, axis=1, keepdims=True)


def _poff(cnt):
    return pl.pallas_call(
        _poff_body,
        grid=(1,),
        in_specs=[pl.BlockSpec((8, 128), lambda i: (0, 0))],
        out_specs=[pl.BlockSpec((8, 128), lambda i: (0, 0)),
                   pl.BlockSpec((8, NTILES), lambda i: (0, 0))],
        out_shape=[jax.ShapeDtypeStruct((8, 128), jnp.float32),
                   jax.ShapeDtypeStruct((8, NTILES), jnp.float32)],
    )(cnt)


def _dest_body(ip_ref, rank_ref, poff_ref, dest_ref):
    erange = lax.broadcasted_iota(jnp.int32, (1, E), 1)
    lane = lax.broadcasted_iota(jnp.int32, (SB, 128), 1)
    poff = poff_ref[0:1, 0:E]
    add = jnp.zeros((SB, 128), jnp.float32)
    for k in range(TOPK):
        ek = ip_ref[:, k:k + 1]
        oh = (ek == erange).astype(jnp.float32)
        pe = jnp.sum(oh * poff, axis=1, keepdims=True)
        add = jnp.where(lane == k, pe, add)
    dest_ref[...] = rank_ref[...] + add


def _dest(ipad, rankpad, poff):
    return pl.pallas_call(
        _dest_body,
        grid=(NSB,),
        in_specs=[pl.BlockSpec((SB, 128), lambda i: (i, 0)),
                  pl.BlockSpec((SB, 128), lambda i: (i, 0)),
                  pl.BlockSpec((8, 128), lambda i: (0, 0))],
        out_specs=pl.BlockSpec((SB, 128), lambda i: (i, 0)),
        out_shape=jax.ShapeDtypeStruct((S, 128), jnp.float32),
    )(ipad, rankpad, poff)


def _wcast_body(gup_ref, dwn_ref, gub_ref, dwb_ref):
    gub_ref[...] = gup_ref[...].astype(jnp.bfloat16)
    dwb_ref[...] = dwn_ref[...].astype(jnp.bfloat16)


def _wcast(gate_up_w, down_w):
    return pl.pallas_call(
        _wcast_body,
        grid=(E, 2),
        in_specs=[pl.BlockSpec((1, M, D), lambda e, j: (e, j, 0)),
                  pl.BlockSpec((1, M, M), lambda e, j: (e, j, 0))],
        out_specs=[pl.BlockSpec((1, M, D), lambda e, j: (e, j, 0)),
                   pl.BlockSpec((1, M, M), lambda e, j: (e, j, 0))],
        out_shape=[jax.ShapeDtypeStruct((E, 2 * M, D), jnp.bfloat16),
                   jax.ShapeDtypeStruct((E, D, M), jnp.bfloat16)],
    )(gate_up_w, down_w)


def _ffn_body(ntu_ref, te_ref, xg_ref, gup_ref, dwn_ref, o_ref):
    @pl.when(pl.program_id(0) < ntu_ref[0])
    def _():
        x = xg_ref[...]
        gu = lax.dot_general(x, gup_ref[0], (((1,), (1,)), ((), ())),
                             preferred_element_type=jnp.float32)
        g = gu[:, :M]
        u = gu[:, M:]
        act = g * (1.0 / (1.0 + jnp.exp(-g))) * u
        o_ref[...] = lax.dot_general(act, dwn_ref[0], (((1,), (1,)), ((), ())),
                                     preferred_element_type=jnp.float32)


def _ffn(ntu, te, xg, gate_up_w, down_w):
    grid_spec = pltpu.PrefetchScalarGridSpec(
        num_scalar_prefetch=2,
        grid=(NTILES,),
        in_specs=[
            pl.BlockSpec((TILE, D), lambda i, ntu, te: (i, 0)),
            pl.BlockSpec((1, 2 * M, D), lambda i, ntu, te: (te[i], 0, 0)),
            pl.BlockSpec((1, D, M), lambda i, ntu, te: (te[i], 0, 0)),
        ],
        out_specs=pl.BlockSpec((TILE, D), lambda i, ntu, te: (i, 0)),
    )
    return pl.pallas_call(
        _ffn_body,
        grid_spec=grid_spec,
        out_shape=jax.ShapeDtypeStruct((PN, D), jnp.float32),
    )(ntu, te, xg, gate_up_w, down_w)


def _comb_body(hg_ref, wp_ref, x2_ref, o_ref):
    k = pl.program_id(1)
    lane = lax.broadcasted_iota(jnp.int32, (SB, 128), 1)
    wk = jnp.sum(jnp.where(lane == k, wp_ref[...], 0.0), axis=1, keepdims=True)
    contrib = hg_ref[...] * wk

    @pl.when(k == 0)
    def _():
        o_ref[...] = x2_ref[...] + contrib

    @pl.when(k > 0)
    def _():
        o_ref[...] = o_ref[...] + contrib


def _combine(hg, wpad, x2):
    return pl.pallas_call(
        _comb_body,
        grid=(NSB, TOPK),
        in_specs=[pl.BlockSpec((SB, D), lambda i, k: (k * NSB + i, 0)),
                  pl.BlockSpec((SB, 128), lambda i, k: (i, 0)),
                  pl.BlockSpec((SB, D), lambda i, k: (i, 0))],
        out_specs=pl.BlockSpec((SB, D), lambda i, k: (i, 0)),
        out_shape=jax.ShapeDtypeStruct((S, D), jnp.float32),
    )(hg, wpad, x2)


_NW = 32


def _sc_dispatch(x3, idx4):
    mesh = plsc.VectorSubcoreMesh(core_axis_name="c", subcore_axis_name="s")
    tok_per_w = S // _NW
    tchunk = 16
    nchunk = tok_per_w // tchunk

    @functools.partial(
        pl.kernel, mesh=mesh,
        out_type=jax.ShapeDtypeStruct((PN, D), jnp.float32),
        scratch_types=[pltpu.VMEM((nchunk, TOPK, tchunk), jnp.int32),
                       pltpu.VMEM((tchunk, D), jnp.float32),
                       pltpu.VMEM((tchunk, D), jnp.float32),
                       pltpu.SemaphoreType.DMA,
                       pltpu.SemaphoreType.DMA],
    )
    def k(x3_hbm, idx_hbm, out_hbm, idx_v, buf0, buf1, rsem, ssem):
        wid = lax.axis_index("s") * 2 + lax.axis_index("c")
        tok_base = wid * tok_per_w
        bufs = (buf0, buf1)
        pltpu.sync_copy(idx_hbm.at[wid], idx_v)
        pltpu.async_copy(x3_hbm.at[pl.ds(tok_base, tchunk)], buf0, rsem)
        for c in range(nchunk):
            buf = bufs[c % 2]
            pltpu.make_async_copy(x3_hbm.at[pl.ds(0, tchunk)], buf,
                                  rsem).wait()
            if c + 1 < nchunk:
                pltpu.async_copy(
                    x3_hbm.at[pl.ds(tok_base + (c + 1) * tchunk, tchunk)],
                    bufs[(c + 1) % 2], rsem)
            for kk in range(TOPK):
                pltpu.async_copy(buf, out_hbm.at[idx_v.at[c, kk]], ssem)
            for kk in range(TOPK):
                pltpu.make_async_copy(x3_hbm.at[pl.ds(0, tchunk)], buf,
                                      ssem).wait()
    return k(x3, idx4)


def _sc_gather_rows(table, idx, nrows, clamp_max):
    mesh = plsc.VectorSubcoreMesh(core_axis_name="c", subcore_axis_name="s")
    rows_per_w = nrows // _NW
    chunk = 24 if rows_per_w % 24 == 0 else 16
    nchunk = rows_per_w // chunk
    dt = table.dtype
    ncol = table.shape[1]

    @functools.partial(
        pl.kernel, mesh=mesh,
        out_type=jax.ShapeDtypeStruct((nrows, ncol), dt),
        scratch_types=[pltpu.VMEM((rows_per_w,), jnp.int32),
                       pltpu.VMEM((chunk, ncol), dt),
                       pltpu.VMEM((chunk, ncol), dt),
                       pltpu.SemaphoreType.DMA,
                       pltpu.SemaphoreType.DMA,
                       pltpu.SemaphoreType.DMA,
                       pltpu.SemaphoreType.DMA],
    )
    def k(tab_hbm, idx_hbm, out_hbm, idx_v, buf0, buf1, gs0, gs1, ws0, ws1):
        wid = lax.axis_index("s") * 2 + lax.axis_index("c")
        base = wid * rows_per_w
        bufs = (buf0, buf1)
        gsems = (gs0, gs1)
        wsems = (ws0, ws1)

        pltpu.sync_copy(idx_hbm.at[pl.ds(base, rows_per_w)], idx_v)
        for h in range(rows_per_w // 16):
            v = idx_v[pl.ds(h * 16, 16)]
            idx_v[pl.ds(h * 16, 16)] = jnp.minimum(jnp.maximum(v, 0), clamp_max)

        def gstart(c, b):
            pltpu.async_copy(tab_hbm.at[idx_v.at[pl.ds(c * chunk, chunk)]],
                             bufs[b], gsems[b])

        def gwait(b):
            pltpu.make_async_copy(tab_hbm.at[pl.ds(0, chunk)], bufs[b],
                                  gsems[b]).wait()

        def wstart(c, b):
            pltpu.async_copy(bufs[b], out_hbm.at[pl.ds(base + c * chunk, chunk)],
                             wsems[b])

        def wwait(b):
            pltpu.make_async_copy(tab_hbm.at[pl.ds(0, chunk)], bufs[b],
                                  wsems[b]).wait()

        gstart(0, 0)

        def step(c, carry):
            even = c % 2 == 0

            @pl.when(c >= 1)
            def _():
                @pl.when(even)
                def _():
                    wwait(1)

                @pl.when(jnp.logical_not(even))
                def _():
                    wwait(0)

            @pl.when(c + 1 < nchunk)
            def _():
                @pl.when(even)
                def _():
                    gstart(c + 1, 1)

                @pl.when(jnp.logical_not(even))
                def _():
                    gstart(c + 1, 0)

            @pl.when(even)
            def _():
                gwait(0)
                wstart(c, 0)

            @pl.when(jnp.logical_not(even))
            def _():
                gwait(1)
                wstart(c, 1)

            return carry

        lax.fori_loop(0, nchunk, step, 0)
        wwait((nchunk - 1) % 2)

    return k(table, idx)


def kernel(x, ln1_w, q_w, k_w, v_w, qn_w, kn_w, o_w, ln2_w, gate_w, gate_up_w, down_w):
    x2d = x.reshape(S, D)

    xn = _rmsnorm(x2d, ln1_w)
    wqkv = jnp.concatenate([q_w, k_w, v_w], axis=0)
    qkv = _qkv_proj(xn, wqkv)
    q3 = qkv[:, :D].reshape(S, H, HD)
    k3 = qkv[:, D:2 * D].reshape(S, H, HD)
    v3 = qkv[:, 2 * D:].reshape(S, H, HD)

    qr = _rope(q3, qn_w.reshape(1, H, HD))
    kr = _rope(k3, kn_w.reshape(1, H, HD))

    qh = qr.transpose(1, 0, 2)
    kh = kr.transpose(1, 0, 2)
    vh = v3.transpose(1, 0, 2)
    ctxh = _attention(qh, kh, vh)
    ctx = ctxh.transpose(1, 0, 2).reshape(S, D)

    x2 = _oproj_res(ctx, o_w, x2d)
    x3, wpad, ipad = _router(x2, ln2_w, gate_w)

    rankpad, cnt = _meta(ipad)
    poff, tef = _poff(cnt)
    destpad = _dest(ipad, rankpad, poff)

    te = tef[0, :].astype(jnp.int32)
    dest_i = destpad[:, :TOPK].astype(jnp.int32)
    idx4 = dest_i.reshape(_NW, 4, 16, TOPK).transpose(0, 1, 3, 2)

    ntu = tef[1, 0:1].astype(jnp.int32)
    xg = _sc_dispatch(x3, idx4)
    h_sorted = _ffn(ntu, te, xg, gate_up_w, down_w)

    cidx = dest_i.T.reshape(NA)
    hg = _sc_gather_rows(h_sorted, cidx, NA, PN - 1)

    out = _combine(hg, wpad, x2)
    return out.reshape(1, S, D)


# speedup vs baseline: 1.7768x; 1.1791x over previous
import functools

import jax
import jax.numpy as jnp
from jax import lax
from jax.experimental import pallas as pl
from jax.experimental.pallas import tpu as pltpu
from jax.experimental.pallas import tpu_sc as plsc

D = 2048
H = 16
HD = 128
E = 64
TOPK = 8
M = 1024
SCALE = 0.08838834764831845
S = 2048
EPS = 1e-05

TILE = 256
PN = S * TOPK + E * TILE
NTILES = PN // TILE
NA = S * TOPK

SB = 256
NSB = S // SB


def _norm1_body(x_ref, w_ref, o_ref):
    x = x_ref[...]
    o_ref[...] = x * lax.rsqrt(jnp.mean(x * x, axis=-1, keepdims=True) + EPS) * w_ref[...]


def _rmsnorm(x2d, w):
    return pl.pallas_call(
        _norm1_body,
        grid=(NSB,),
        in_specs=[pl.BlockSpec((SB, D), lambda i: (i, 0)),
                  pl.BlockSpec((1, D), lambda i: (0, 0))],
        out_specs=pl.BlockSpec((SB, D), lambda i: (i, 0)),
        out_shape=jax.ShapeDtypeStruct((S, D), jnp.float32),
    )(x2d, w.reshape(1, D))


def _qkv_body(xn_ref, w_ref, o_ref):
    o_ref[...] = lax.dot_general(xn_ref[...], w_ref[...],
                                 (((1,), (1,)), ((), ())),
                                 preferred_element_type=jnp.float32)


def _qkv_proj(xn, wqkv):
    return pl.pallas_call(
        _qkv_body,
        grid=(6, NSB),
        in_specs=[pl.BlockSpec((SB, D), lambda j, i: (i, 0)),
                  pl.BlockSpec((1024, D), lambda j, i: (j, 0))],
        out_specs=pl.BlockSpec((SB, 1024), lambda j, i: (i, j)),
        out_shape=jax.ShapeDtypeStruct((S, 3 * D), jnp.float32),
    )(xn, wqkv)


def _rope_body(q_ref, w_ref, o_ref):
    q = q_ref[...]
    blk = q.shape[0]
    qn = q * lax.rsqrt(jnp.mean(q * q, axis=(1, 2), keepdims=True) + EPS) * w_ref[...]
    i = pl.program_id(0)
    pos = (i * blk + lax.broadcasted_iota(jnp.int32, (blk, 1, 1), 0)).astype(jnp.float32)
    lanes = lax.broadcasted_iota(jnp.int32, (1, 1, HD), 2)
    f = (lanes % 64).astype(jnp.float32)
    inv_freq = jnp.exp(f * (-jnp.log(10000.0) / 64.0))
    ang = pos * inv_freq
    c, s = jnp.cos(ang), jnp.sin(ang)
    a = qn * c
    b = qn * s
    b_sw = pltpu.roll(b, 64, 2)
    o_ref[...] = a + jnp.where(lanes < 64, -b_sw, b_sw)


def _rope(q3, w3):
    return pl.pallas_call(
        _rope_body,
        grid=(16,),
        in_specs=[pl.BlockSpec((128, H, HD), lambda i: (i, 0, 0)),
                  pl.BlockSpec((1, H, HD), lambda i: (0, 0, 0))],
        out_specs=pl.BlockSpec((128, H, HD), lambda i: (i, 0, 0)),
        out_shape=jax.ShapeDtypeStruct((S, H, HD), jnp.float32),
    )(q3, w3)


def _attn_body(q_ref, k_ref, v_ref, o_ref):
    i = pl.program_id(1)
    q = q_ref[0]
    k = k_ref[0]
    v = v_ref[0]
    s = lax.dot_general(q, k, (((1,), (1,)), ((), ())),
                        preferred_element_type=jnp.float32) * SCALE
    rows = i * SB + lax.broadcasted_iota(jnp.int32, (SB, S), 0)
    cols = lax.broadcasted_iota(jnp.int32, (SB, S), 1)
    s = jnp.where(cols <= rows, s, -jnp.inf)
    m = jnp.max(s, axis=1, keepdims=True)
    p = jnp.exp(s - m)
    z = jnp.sum(p, axis=1, keepdims=True)
    o_ref[0] = lax.dot_general(p, v, (((1,), (0,)), ((), ())),
                               preferred_element_type=jnp.float32) / z


def _attention(qh, kh, vh):
    return pl.pallas_call(
        _attn_body,
        grid=(H, NSB),
        in_specs=[pl.BlockSpec((1, SB, HD), lambda h, i: (h, i, 0)),
                  pl.BlockSpec((1, S, HD), lambda h, i: (h, 0, 0)),
                  pl.BlockSpec((1, S, HD), lambda h, i: (h, 0, 0))],
        out_specs=pl.BlockSpec((1, SB, HD), lambda h, i: (h, i, 0)),
        out_shape=jax.ShapeDtypeStruct((H, S, HD), jnp.float32),
    )(qh, kh, vh)


def _oproj_body(c_ref, w_ref, x_ref, o_ref):
    o_ref[...] = lax.dot_general(c_ref[...], w_ref[...],
                                 (((1,), (1,)), ((), ())),
                                 preferred_element_type=jnp.float32) + x_ref[...]


def _oproj_res(ctx, o_w, x2d):
    return pl.pallas_call(
        _oproj_body,
        grid=(4, NSB),
        in_specs=[pl.BlockSpec((SB, D), lambda j, i: (i, 0)),
                  pl.BlockSpec((512, D), lambda j, i: (j, 0)),
                  pl.BlockSpec((SB, 512), lambda j, i: (i, j))],
        out_specs=pl.BlockSpec((SB, 512), lambda j, i: (i, j)),
        out_shape=jax.ShapeDtypeStruct((S, D), jnp.float32),
    )(ctx, o_w, x2d)


def _router_body(x2_ref, ln2_ref, gw_ref, x3_ref, wp_ref, ip_ref):
    x2 = x2_ref[...]
    x3 = x2 * lax.rsqrt(jnp.mean(x2 * x2, axis=-1, keepdims=True) + EPS) * ln2_ref[...]
    x3_ref[...] = x3
    logits = lax.dot_general(x3, gw_ref[...], (((1,), (1,)), ((), ())),
                             preferred_element_type=jnp.float32)
    mx = jnp.max(logits, axis=1, keepdims=True)
    ex = jnp.exp(logits - mx)
    probs = ex / jnp.sum(ex, axis=1, keepdims=True)
    cols = lax.broadcasted_iota(jnp.int32, (SB, E), 1)
    lane = lax.broadcasted_iota(jnp.int32, (SB, 128), 1)
    accw = jnp.zeros((SB, 128), jnp.float32)
    acci = jnp.zeros((SB, 128), jnp.int32)
    for j in range(TOPK):
        m = jnp.max(probs, axis=1, keepdims=True)
        idx = jnp.min(jnp.where(probs == m, cols, E), axis=1, keepdims=True)
        accw = jnp.where(lane == j, m, accw)
        acci = jnp.where(lane == j, idx, acci)
        probs = jnp.where(cols == idx, -1.0, probs)
    wp_ref[...] = accw
    ip_ref[...] = acci


def _router(x2, ln2_w, gate_w):
    return pl.pallas_call(
        _router_body,
        grid=(NSB,),
        in_specs=[pl.BlockSpec((SB, D), lambda i: (i, 0)),
                  pl.BlockSpec((1, D), lambda i: (0, 0)),
                  pl.BlockSpec((E, D), lambda i: (0, 0))],
        out_specs=[pl.BlockSpec((SB, D), lambda i: (i, 0)),
                   pl.BlockSpec((SB, 128), lambda i: (i, 0)),
                   pl.BlockSpec((SB, 128), lambda i: (i, 0))],
        out_shape=[jax.ShapeDtypeStruct((S, D), jnp.float32),
                   jax.ShapeDtypeStruct((S, 128), jnp.float32),
                   jax.ShapeDtypeStruct((S, 128), jnp.int32)],
    )(x2, ln2_w.reshape(1, D), gate_w)


def _meta_body(ip_ref, rank_ref, cnt_ref, carry):
    i = pl.program_id(0)

    @pl.when(i == 0)
    def _():
        carry[...] = jnp.zeros_like(carry)

    erange = lax.broadcasted_iota(jnp.int32, (1, E), 1)
    ohs = []
    for k in range(TOPK):
        ek = ip_ref[:, k:k + 1]
        ohs.append((ek == erange).astype(jnp.float32))
    rowsum = sum(ohs)
    r = lax.broadcasted_iota(jnp.int32, (SB, SB), 0)
    c = lax.broadcasted_iota(jnp.int32, (SB, SB), 1)
    ltri = (c < r).astype(jnp.float32)
    rowpre = lax.dot_general(ltri, rowsum, (((1,), (0,)), ((), ())),
                             preferred_element_type=jnp.float32)
    acc = rowpre + carry[0:1, 0:E]
    lane = lax.broadcasted_iota(jnp.int32, (SB, 128), 1)
    rk = jnp.zeros((SB, 128), jnp.float32)
    for k in range(TOPK):
        rkk = jnp.sum(ohs[k] * acc, axis=1, keepdims=True)
        rk = jnp.where(lane == k, rkk, rk)
        acc = acc + ohs[k]
    rank_ref[...] = rk
    carry[0:1, 0:E] = carry[0:1, 0:E] + jnp.sum(rowsum, axis=0, keepdims=True)
    cnt_ref[...] = jnp.broadcast_to(carry[0:1, :], (8, 128))


def _meta(ipad):
    return pl.pallas_call(
        _meta_body,
        grid=(NSB,),
        in_specs=[pl.BlockSpec((SB, 128), lambda i: (i, 0))],
        out_specs=[pl.BlockSpec((SB, 128), lambda i: (i, 0)),
                   pl.BlockSpec((8, 128), lambda i: (0, 0))],
        out_shape=[jax.ShapeDtypeStruct((S, 128), jnp.float32),
                   jax.ShapeDtypeStruct((8, 128), jnp.float32)],
        scratch_shapes=[pltpu.VMEM((8, 128), jnp.float32)],
    )(ipad)


def _poff_body(cnt_ref, poff_ref, te_ref):
    c = cnt_ref[0:1, 0:E]
    nt = jnp.floor((c + (TILE - 1.0)) * (1.0 / TILE))
    r = lax.broadcasted_iota(jnp.int32, (E, E), 0)
    cc = lax.broadcasted_iota(jnp.int32, (E, E), 1)
    utri = (r < cc).astype(jnp.float32)
    ts = lax.dot_general(nt, utri, (((1,), (0,)), ((), ())),
                         preferred_element_type=jnp.float32)
    poff_ref[...] = jnp.zeros((8, 128), jnp.float32)
    poff_ref[0:1, 0:E] = ts * TILE
    ts_col = jnp.transpose(ts, (1, 0))
    ti = lax.broadcasted_iota(jnp.int32, (1, NTILES), 1).astype(jnp.float32)
    te = jnp.sum((ti >= ts_col).astype(jnp.float32), axis=0, keepdims=True) - 1.0
    te = jnp.clip(te, 0.0, E - 1.0)
    te_ref[...] = jnp.broadcast_to(te, (8, NTILES))
    te_ref[1:2, 0:1] = jnp.sum(nt, axis=1, keepdims=True)


def _poff(cnt):
    return pl.pallas_call(
        _poff_body,
        grid=(1,),
        in_specs=[pl.BlockSpec((8, 128), lambda i: (0, 0))],
        out_specs=[pl.BlockSpec((8, 128), lambda i: (0, 0)),
                   pl.BlockSpec((8, NTILES), lambda i: (0, 0))],
        out_shape=[jax.ShapeDtypeStruct((8, 128), jnp.float32),
                   jax.ShapeDtypeStruct((8, NTILES), jnp.float32)],
    )(cnt)


def _dest_body(ip_ref, rank_ref, poff_ref, dest_ref):
    erange = lax.broadcasted_iota(jnp.int32, (1, E), 1)
    lane = lax.broadcasted_iota(jnp.int32, (SB, 128), 1)
    poff = poff_ref[0:1, 0:E]
    add = jnp.zeros((SB, 128), jnp.float32)
    for k in range(TOPK):
        ek = ip_ref[:, k:k + 1]
        oh = (ek == erange).astype(jnp.float32)
        pe = jnp.sum(oh * poff, axis=1, keepdims=True)
        add = jnp.where(lane == k, pe, add)
    dest_ref[...] = rank_ref[...] + add


def _dest(ipad, rankpad, poff):
    return pl.pallas_call(
        _dest_body,
        grid=(NSB,),
        in_specs=[pl.BlockSpec((SB, 128), lambda i: (i, 0)),
                  pl.BlockSpec((SB, 128), lambda i: (i, 0)),
                  pl.BlockSpec((8, 128), lambda i: (0, 0))],
        out_specs=pl.BlockSpec((SB, 128), lambda i: (i, 0)),
        out_shape=jax.ShapeDtypeStruct((S, 128), jnp.float32),
    )(ipad, rankpad, poff)


def _wcast_body(gup_ref, dwn_ref, gub_ref, dwb_ref):
    gub_ref[...] = gup_ref[...].astype(jnp.bfloat16)
    dwb_ref[...] = dwn_ref[...].astype(jnp.bfloat16)


def _wcast(gate_up_w, down_w):
    return pl.pallas_call(
        _wcast_body,
        grid=(E, 2),
        in_specs=[pl.BlockSpec((1, M, D), lambda e, j: (e, j, 0)),
                  pl.BlockSpec((1, M, M), lambda e, j: (e, j, 0))],
        out_specs=[pl.BlockSpec((1, M, D), lambda e, j: (e, j, 0)),
                   pl.BlockSpec((1, M, M), lambda e, j: (e, j, 0))],
        out_shape=[jax.ShapeDtypeStruct((E, 2 * M, D), jnp.bfloat16),
                   jax.ShapeDtypeStruct((E, D, M), jnp.bfloat16)],
    )(gate_up_w, down_w)


def _ffn_body(ntu_ref, te_ref, xg_ref, gup_ref, dwn_ref, o_ref):
    @pl.when(pl.program_id(0) < ntu_ref[0])
    def _():
        x = xg_ref[...]
        gu = lax.dot_general(x, gup_ref[0], (((1,), (1,)), ((), ())),
                             preferred_element_type=jnp.float32)
        g = gu[:, :M]
        u = gu[:, M:]
        act = g * (1.0 / (1.0 + jnp.exp(-g))) * u
        o_ref[...] = lax.dot_general(act, dwn_ref[0], (((1,), (1,)), ((), ())),
                                     preferred_element_type=jnp.float32)


def _ffn(ntu, te, xg, gate_up_w, down_w):
    grid_spec = pltpu.PrefetchScalarGridSpec(
        num_scalar_prefetch=2,
        grid=(NTILES,),
        in_specs=[
            pl.BlockSpec((TILE, D), lambda i, ntu, te: (i, 0)),
            pl.BlockSpec((1, 2 * M, D), lambda i, ntu, te: (te[i], 0, 0)),
            pl.BlockSpec((1, D, M), lambda i, ntu, te: (te[i], 0, 0)),
        ],
        out_specs=pl.BlockSpec((TILE, D), lambda i, ntu, te: (i, 0)),
    )
    return pl.pallas_call(
        _ffn_body,
        grid_spec=grid_spec,
        out_shape=jax.ShapeDtypeStruct((PN, D), jnp.float32),
    )(ntu, te, xg, gate_up_w, down_w)


def _comb_body(hg_ref, wp_ref, x2_ref, o_ref):
    k = pl.program_id(1)
    lane = lax.broadcasted_iota(jnp.int32, (SB, 128), 1)
    wk = jnp.sum(jnp.where(lane == k, wp_ref[...], 0.0), axis=1, keepdims=True)
    contrib = hg_ref[...] * wk

    @pl.when(k == 0)
    def _():
        o_ref[...] = x2_ref[...] + contrib

    @pl.when(k > 0)
    def _():
        o_ref[...] = o_ref[...] + contrib


def _combine(hg, wpad, x2):
    return pl.pallas_call(
        _comb_body,
        grid=(NSB, TOPK),
        in_specs=[pl.BlockSpec((SB, D), lambda i, k: (k * NSB + i, 0)),
                  pl.BlockSpec((SB, 128), lambda i, k: (i, 0)),
                  pl.BlockSpec((SB, D), lambda i, k: (i, 0))],
        out_specs=pl.BlockSpec((SB, D), lambda i, k: (i, 0)),
        out_shape=jax.ShapeDtypeStruct((S, D), jnp.float32),
    )(hg, wpad, x2)


_NW = 32


def _sc_dispatch(x3, idx4):
    mesh = plsc.VectorSubcoreMesh(core_axis_name="c", subcore_axis_name="s")
    tok_per_w = S // _NW
    tchunk = 16
    nchunk = tok_per_w // tchunk

    @functools.partial(
        pl.kernel, mesh=mesh,
        out_type=jax.ShapeDtypeStruct((PN, D), jnp.float32),
        scratch_types=[pltpu.VMEM((nchunk, TOPK, tchunk), jnp.int32),
                       pltpu.VMEM((tchunk, D), jnp.float32),
                       pltpu.VMEM((tchunk, D), jnp.float32),
                       pltpu.SemaphoreType.DMA,
                       pltpu.SemaphoreType.DMA],
    )
    def k(x3_hbm, idx_hbm, out_hbm, idx_v, buf0, buf1, rsem, ssem):
        wid = lax.axis_index("s") * 2 + lax.axis_index("c")
        tok_base = wid * tok_per_w
        bufs = (buf0, buf1)
        pltpu.sync_copy(idx_hbm.at[wid], idx_v)
        pltpu.async_copy(x3_hbm.at[pl.ds(tok_base, tchunk)], buf0, rsem)
        for c in range(nchunk):
            buf = bufs[c % 2]
            pltpu.make_async_copy(x3_hbm.at[pl.ds(0, tchunk)], buf,
                                  rsem).wait()
            if c + 1 < nchunk:
                pltpu.async_copy(
                    x3_hbm.at[pl.ds(tok_base + (c + 1) * tchunk, tchunk)],
                    bufs[(c + 1) % 2], rsem)
            for kk in range(TOPK):
                pltpu.async_copy(buf, out_hbm.at[idx_v.at[c, kk]], ssem)
            for kk in range(TOPK):
                pltpu.make_async_copy(x3_hbm.at[pl.ds(0, tchunk)], buf,
                                      ssem).wait()
    return k(x3, idx4)


def _sc_gather_rows(table, idx, nrows, clamp_max):
    mesh = plsc.VectorSubcoreMesh(core_axis_name="c", subcore_axis_name="s")
    rows_per_w = nrows // _NW
    chunk = 24 if rows_per_w % 24 == 0 else 16
    nchunk = rows_per_w // chunk
    dt = table.dtype
    ncol = table.shape[1]

    @functools.partial(
        pl.kernel, mesh=mesh,
        out_type=jax.ShapeDtypeStruct((nrows, ncol), dt),
        scratch_types=[pltpu.VMEM((rows_per_w,), jnp.int32),
                       pltpu.VMEM((chunk, ncol), dt),
                       pltpu.VMEM((chunk, ncol), dt),
                       pltpu.SemaphoreType.DMA,
                       pltpu.SemaphoreType.DMA,
                       pltpu.SemaphoreType.DMA,
                       pltpu.SemaphoreType.DMA],
    )
    def k(tab_hbm, idx_hbm, out_hbm, idx_v, buf0, buf1, gs0, gs1, ws0, ws1):
        wid = lax.axis_index("s") * 2 + lax.axis_index("c")
        base = wid * rows_per_w
        bufs = (buf0, buf1)
        gsems = (gs0, gs1)
        wsems = (ws0, ws1)

        pltpu.sync_copy(idx_hbm.at[pl.ds(base, rows_per_w)], idx_v)
        for h in range(rows_per_w // 16):
            v = idx_v[pl.ds(h * 16, 16)]
            idx_v[pl.ds(h * 16, 16)] = jnp.minimum(jnp.maximum(v, 0), clamp_max)

        def gstart(c, b):
            pltpu.async_copy(tab_hbm.at[idx_v.at[pl.ds(c * chunk, chunk)]],
                             bufs[b], gsems[b])

        def gwait(b):
            pltpu.make_async_copy(tab_hbm.at[pl.ds(0, chunk)], bufs[b],
                                  gsems[b]).wait()

        def wstart(c, b):
            pltpu.async_copy(bufs[b], out_hbm.at[pl.ds(base + c * chunk, chunk)],
                             wsems[b])

        def wwait(b):
            pltpu.make_async_copy(tab_hbm.at[pl.ds(0, chunk)], bufs[b],
                                  wsems[b]).wait()

        gstart(0, 0)

        def step(c, carry):
            even = c % 2 == 0

            @pl.when(c >= 1)
            def _():
                @pl.when(even)
                def _():
                    wwait(1)

                @pl.when(jnp.logical_not(even))
                def _():
                    wwait(0)

            @pl.when(c + 1 < nchunk)
            def _():
                @pl.when(even)
                def _():
                    gstart(c + 1, 1)

                @pl.when(jnp.logical_not(even))
                def _():
                    gstart(c + 1, 0)

            @pl.when(even)
            def _():
                gwait(0)
                wstart(c, 0)

            @pl.when(jnp.logical_not(even))
            def _():
                gwait(1)
                wstart(c, 1)

            return carry

        lax.fori_loop(0, nchunk, step, 0)
        wwait((nchunk - 1) % 2)

    return k(table, idx)


def kernel(x, ln1_w, q_w, k_w, v_w, qn_w, kn_w, o_w, ln2_w, gate_w, gate_up_w, down_w):
    x2d = x.reshape(S, D)

    xn = _rmsnorm(x2d, ln1_w)
    wqkv = jnp.concatenate([q_w, k_w, v_w], axis=0)
    qkv = _qkv_proj(xn, wqkv)
    q3 = qkv[:, :D].reshape(S, H, HD)
    k3 = qkv[:, D:2 * D].reshape(S, H, HD)
    v3 = qkv[:, 2 * D:].reshape(S, H, HD)

    qr = _rope(q3, qn_w.reshape(1, H, HD))
    kr = _rope(k3, kn_w.reshape(1, H, HD))

    qh = qr.transpose(1, 0, 2)
    kh = kr.transpose(1, 0, 2)
    vh = v3.transpose(1, 0, 2)
    ctxh = _attention(qh, kh, vh)
    ctx = ctxh.transpose(1, 0, 2).reshape(S, D)

    x2 = _oproj_res(ctx, o_w, x2d)
    x3, wpad, ipad = _router(x2, ln2_w, gate_w)

    rankpad, cnt = _meta(ipad)
    poff, tef = _poff(cnt)
    destpad = _dest(ipad, rankpad, poff)

    te = tef[0, :].astype(jnp.int32)
    dest_i = destpad[:, :TOPK].astype(jnp.int32)
    idx4 = dest_i.reshape(_NW, 4, 16, TOPK).transpose(0, 1, 3, 2)

    ntu = tef[1, 0:1].astype(jnp.int32)
    xg = _sc_dispatch(x3, idx4)
    h_sorted = _ffn(ntu, te, xg, gate_up_w, down_w)

    cidx = dest_i.T.reshape(NA)
    hg = _sc_gather_rows(h_sorted, cidx, NA, PN - 1)

    out = _combine(hg, wpad, x2)
    return out.reshape(1, S, D)


# speedup vs baseline: 1.8560x; 1.0446x over previous
import functools

import jax
import jax.numpy as jnp
from jax import lax
from jax.experimental import pallas as pl
from jax.experimental.pallas import tpu as pltpu
from jax.experimental.pallas import tpu_sc as plsc

D = 2048
H = 16
HD = 128
E = 64
TOPK = 8
M = 1024
SCALE = 0.08838834764831845
S = 2048
EPS = 1e-05

TILE = 256
PN = S * TOPK + E * TILE
NTILES = PN // TILE
NA = S * TOPK

SB = 256
NSB = S // SB


def _norm1_body(x_ref, w_ref, o_ref):
    x = x_ref[...]
    o_ref[...] = x * lax.rsqrt(jnp.mean(x * x, axis=-1, keepdims=True) + EPS) * w_ref[...]


def _rmsnorm(x2d, w):
    return pl.pallas_call(
        _norm1_body,
        grid=(NSB,),
        in_specs=[pl.BlockSpec((SB, D), lambda i: (i, 0)),
                  pl.BlockSpec((1, D), lambda i: (0, 0))],
        out_specs=pl.BlockSpec((SB, D), lambda i: (i, 0)),
        out_shape=jax.ShapeDtypeStruct((S, D), jnp.float32),
    )(x2d, w.reshape(1, D))


def _qkv_body(xn_ref, w_ref, o_ref):
    o_ref[...] = lax.dot_general(xn_ref[...].astype(jnp.bfloat16), w_ref[...],
                                 (((1,), (1,)), ((), ())),
                                 preferred_element_type=jnp.float32)


def _qkv_proj(xn, wqkv):
    return pl.pallas_call(
        _qkv_body,
        grid=(6, NSB),
        in_specs=[pl.BlockSpec((SB, D), lambda j, i: (i, 0)),
                  pl.BlockSpec((1024, D), lambda j, i: (j, 0))],
        out_specs=pl.BlockSpec((SB, 1024), lambda j, i: (i, j)),
        out_shape=jax.ShapeDtypeStruct((S, 3 * D), jnp.float32),
    )(xn, wqkv)


def _rope_body(q_ref, w_ref, o_ref):
    q = q_ref[...]
    blk = q.shape[0]
    qn = q * lax.rsqrt(jnp.mean(q * q, axis=(1, 2), keepdims=True) + EPS) * w_ref[...]
    i = pl.program_id(0)
    pos = (i * blk + lax.broadcasted_iota(jnp.int32, (blk, 1, 1), 0)).astype(jnp.float32)
    lanes = lax.broadcasted_iota(jnp.int32, (1, 1, HD), 2)
    f = (lanes % 64).astype(jnp.float32)
    inv_freq = jnp.exp(f * (-jnp.log(10000.0) / 64.0))
    ang = pos * inv_freq
    c, s = jnp.cos(ang), jnp.sin(ang)
    a = qn * c
    b = qn * s
    b_sw = pltpu.roll(b, 64, 2)
    o_ref[...] = a + jnp.where(lanes < 64, -b_sw, b_sw)


def _rope(q3, w3):
    return pl.pallas_call(
        _rope_body,
        grid=(16,),
        in_specs=[pl.BlockSpec((128, H, HD), lambda i: (i, 0, 0)),
                  pl.BlockSpec((1, H, HD), lambda i: (0, 0, 0))],
        out_specs=pl.BlockSpec((128, H, HD), lambda i: (i, 0, 0)),
        out_shape=jax.ShapeDtypeStruct((S, H, HD), jnp.float32),
    )(q3, w3)


def _attn_body(q_ref, k_ref, v_ref, o_ref):
    i = pl.program_id(1)
    q = q_ref[0]
    k = k_ref[0]
    v = v_ref[0]
    s = lax.dot_general(q, k, (((1,), (1,)), ((), ())),
                        preferred_element_type=jnp.float32) * SCALE
    rows = i * SB + lax.broadcasted_iota(jnp.int32, (SB, S), 0)
    cols = lax.broadcasted_iota(jnp.int32, (SB, S), 1)
    s = jnp.where(cols <= rows, s, -jnp.inf)
    m = jnp.max(s, axis=1, keepdims=True)
    p = jnp.exp(s - m)
    z = jnp.sum(p, axis=1, keepdims=True)
    o_ref[0] = lax.dot_general(p.astype(jnp.bfloat16), v,
                               (((1,), (0,)), ((), ())),
                               preferred_element_type=jnp.float32) / z


def _attention(qh, kh, vh):
    return pl.pallas_call(
        _attn_body,
        grid=(H, NSB),
        in_specs=[pl.BlockSpec((1, SB, HD), lambda h, i: (h, i, 0)),
                  pl.BlockSpec((1, S, HD), lambda h, i: (h, 0, 0)),
                  pl.BlockSpec((1, S, HD), lambda h, i: (h, 0, 0))],
        out_specs=pl.BlockSpec((1, SB, HD), lambda h, i: (h, i, 0)),
        out_shape=jax.ShapeDtypeStruct((H, S, HD), jnp.float32),
    )(qh, kh, vh)


def _oproj_body(c_ref, w_ref, x_ref, o_ref):
    o_ref[...] = lax.dot_general(c_ref[...].astype(jnp.bfloat16), w_ref[...],
                                 (((1,), (1,)), ((), ())),
                                 preferred_element_type=jnp.float32) + x_ref[...]


def _oproj_res(ctx, o_w, x2d):
    return pl.pallas_call(
        _oproj_body,
        grid=(4, NSB),
        in_specs=[pl.BlockSpec((SB, D), lambda j, i: (i, 0)),
                  pl.BlockSpec((512, D), lambda j, i: (j, 0)),
                  pl.BlockSpec((SB, 512), lambda j, i: (i, j))],
        out_specs=pl.BlockSpec((SB, 512), lambda j, i: (i, j)),
        out_shape=jax.ShapeDtypeStruct((S, D), jnp.float32),
    )(ctx, o_w, x2d)


def _router_body(x2_ref, ln2_ref, gw_ref, x3_ref, wp_ref, ip_ref):
    x2 = x2_ref[...]
    x3 = x2 * lax.rsqrt(jnp.mean(x2 * x2, axis=-1, keepdims=True) + EPS) * ln2_ref[...]
    x3_ref[...] = x3
    logits = lax.dot_general(x3, gw_ref[...], (((1,), (1,)), ((), ())),
                             preferred_element_type=jnp.float32)
    mx = jnp.max(logits, axis=1, keepdims=True)
    ex = jnp.exp(logits - mx)
    probs = ex / jnp.sum(ex, axis=1, keepdims=True)
    cols = lax.broadcasted_iota(jnp.int32, (SB, E), 1)
    lane = lax.broadcasted_iota(jnp.int32, (SB, 128), 1)
    accw = jnp.zeros((SB, 128), jnp.float32)
    acci = jnp.zeros((SB, 128), jnp.int32)
    for j in range(TOPK):
        m = jnp.max(probs, axis=1, keepdims=True)
        idx = jnp.min(jnp.where(probs == m, cols, E), axis=1, keepdims=True)
        accw = jnp.where(lane == j, m, accw)
        acci = jnp.where(lane == j, idx, acci)
        probs = jnp.where(cols == idx, -1.0, probs)
    wp_ref[...] = accw
    ip_ref[...] = acci


def _router(x2, ln2_w, gate_w):
    return pl.pallas_call(
        _router_body,
        grid=(NSB,),
        in_specs=[pl.BlockSpec((SB, D), lambda i: (i, 0)),
                  pl.BlockSpec((1, D), lambda i: (0, 0)),
                  pl.BlockSpec((E, D), lambda i: (0, 0))],
        out_specs=[pl.BlockSpec((SB, D), lambda i: (i, 0)),
                   pl.BlockSpec((SB, 128), lambda i: (i, 0)),
                   pl.BlockSpec((SB, 128), lambda i: (i, 0))],
        out_shape=[jax.ShapeDtypeStruct((S, D), jnp.float32),
                   jax.ShapeDtypeStruct((S, 128), jnp.float32),
                   jax.ShapeDtypeStruct((S, 128), jnp.int32)],
    )(x2, ln2_w.reshape(1, D), gate_w)


def _meta_body(ip_ref, rank_ref, cnt_ref, carry):
    i = pl.program_id(0)

    @pl.when(i == 0)
    def _():
        carry[...] = jnp.zeros_like(carry)

    erange = lax.broadcasted_iota(jnp.int32, (1, E), 1)
    ohs = []
    for k in range(TOPK):
        ek = ip_ref[:, k:k + 1]
        ohs.append((ek == erange).astype(jnp.float32))
    rowsum = sum(ohs)
    r = lax.broadcasted_iota(jnp.int32, (SB, SB), 0)
    c = lax.broadcasted_iota(jnp.int32, (SB, SB), 1)
    ltri = (c < r).astype(jnp.float32)
    rowpre = lax.dot_general(ltri, rowsum, (((1,), (0,)), ((), ())),
                             preferred_element_type=jnp.float32)
    acc = rowpre + carry[0:1, 0:E]
    lane = lax.broadcasted_iota(jnp.int32, (SB, 128), 1)
    rk = jnp.zeros((SB, 128), jnp.float32)
    for k in range(TOPK):
        rkk = jnp.sum(ohs[k] * acc, axis=1, keepdims=True)
        rk = jnp.where(lane == k, rkk, rk)
        acc = acc + ohs[k]
    rank_ref[...] = rk
    carry[0:1, 0:E] = carry[0:1, 0:E] + jnp.sum(rowsum, axis=0, keepdims=True)
    cnt_ref[...] = jnp.broadcast_to(carry[0:1, :], (8, 128))


def _meta(ipad):
    return pl.pallas_call(
        _meta_body,
        grid=(NSB,),
        in_specs=[pl.BlockSpec((SB, 128), lambda i: (i, 0))],
        out_specs=[pl.BlockSpec((SB, 128), lambda i: (i, 0)),
                   pl.BlockSpec((8, 128), lambda i: (0, 0))],
        out_shape=[jax.ShapeDtypeStruct((S, 128), jnp.float32),
                   jax.ShapeDtypeStruct((8, 128), jnp.float32)],
        scratch_shapes=[pltpu.VMEM((8, 128), jnp.float32)],
    )(ipad)


def _poff_body(cnt_ref, poff_ref, te_ref):
    c = cnt_ref[0:1, 0:E]
    nt = jnp.floor((c + (TILE - 1.0)) * (1.0 / TILE))
    r = lax.broadcasted_iota(jnp.int32, (E, E), 0)
    cc = lax.broadcasted_iota(jnp.int32, (E, E), 1)
    utri = (r < cc).astype(jnp.float32)
    ts = lax.dot_general(nt, utri, (((1,), (0,)), ((), ())),
                         preferred_element_type=jnp.float32)
    poff_ref[...] = jnp.zeros((8, 128), jnp.float32)
    poff_ref[0:1, 0:E] = ts * TILE
    ts_col = jnp.transpose(ts, (1, 0))
    ti = lax.broadcasted_iota(jnp.int32, (1, NTILES), 1).astype(jnp.float32)
    te = jnp.sum((ti >= ts_col).astype(jnp.float32), axis=0, keepdims=True) - 1.0
    te = jnp.clip(te, 0.0, E - 1.0)
    te_ref[...] = jnp.broadcast_to(te, (8, NTILES))
    te_ref[1:2, 0:1] = jnp.sum(nt, axis=1, keepdims=True)


def _poff(cnt):
    return pl.pallas_call(
        _poff_body,
        grid=(1,),
        in_specs=[pl.BlockSpec((8, 128), lambda i: (0, 0))],
        out_specs=[pl.BlockSpec((8, 128), lambda i: (0, 0)),
                   pl.BlockSpec((8, NTILES), lambda i: (0, 0))],
        out_shape=[jax.ShapeDtypeStruct((8, 128), jnp.float32),
                   jax.ShapeDtypeStruct((8, NTILES), jnp.float32)],
    )(cnt)


def _dest_body(ip_ref, rank_ref, poff_ref, dest_ref):
    erange = lax.broadcasted_iota(jnp.int32, (1, E), 1)
    lane = lax.broadcasted_iota(jnp.int32, (SB, 128), 1)
    poff = poff_ref[0:1, 0:E]
    add = jnp.zeros((SB, 128), jnp.float32)
    for k in range(TOPK):
        ek = ip_ref[:, k:k + 1]
        oh = (ek == erange).astype(jnp.float32)
        pe = jnp.sum(oh * poff, axis=1, keepdims=True)
        add = jnp.where(lane == k, pe, add)
    dest_ref[...] = rank_ref[...] + add


def _dest(ipad, rankpad, poff):
    return pl.pallas_call(
        _dest_body,
        grid=(NSB,),
        in_specs=[pl.BlockSpec((SB, 128), lambda i: (i, 0)),
                  pl.BlockSpec((SB, 128), lambda i: (i, 0)),
                  pl.BlockSpec((8, 128), lambda i: (0, 0))],
        out_specs=pl.BlockSpec((SB, 128), lambda i: (i, 0)),
        out_shape=jax.ShapeDtypeStruct((S, 128), jnp.float32),
    )(ipad, rankpad, poff)


def _wcast_body(gup_ref, dwn_ref, gub_ref, dwb_ref):
    gub_ref[...] = gup_ref[...].astype(jnp.bfloat16)
    dwb_ref[...] = dwn_ref[...].astype(jnp.bfloat16)


def _wcast(gate_up_w, down_w):
    return pl.pallas_call(
        _wcast_body,
        grid=(E, 2),
        in_specs=[pl.BlockSpec((1, M, D), lambda e, j: (e, j, 0)),
                  pl.BlockSpec((1, M, M), lambda e, j: (e, j, 0))],
        out_specs=[pl.BlockSpec((1, M, D), lambda e, j: (e, j, 0)),
                   pl.BlockSpec((1, M, M), lambda e, j: (e, j, 0))],
        out_shape=[jax.ShapeDtypeStruct((E, 2 * M, D), jnp.bfloat16),
                   jax.ShapeDtypeStruct((E, D, M), jnp.bfloat16)],
    )(gate_up_w, down_w)


def _ffn_body(ntu_ref, te_ref, xg_ref, gup_ref, dwn_ref, o_ref):
    @pl.when(pl.program_id(0) < ntu_ref[0])
    def _():
        x = xg_ref[...]
        gu = lax.dot_general(x, gup_ref[0], (((1,), (1,)), ((), ())),
                             preferred_element_type=jnp.float32)
        g = gu[:, :M]
        u = gu[:, M:]
        act = g * (1.0 / (1.0 + jnp.exp(-g))) * u
        o_ref[...] = lax.dot_general(act, dwn_ref[0], (((1,), (1,)), ((), ())),
                                     preferred_element_type=jnp.float32)


def _ffn(ntu, te, xg, gate_up_w, down_w):
    grid_spec = pltpu.PrefetchScalarGridSpec(
        num_scalar_prefetch=2,
        grid=(NTILES,),
        in_specs=[
            pl.BlockSpec((TILE, D), lambda i, ntu, te: (i, 0)),
            pl.BlockSpec((1, 2 * M, D), lambda i, ntu, te: (te[i], 0, 0)),
            pl.BlockSpec((1, D, M), lambda i, ntu, te: (te[i], 0, 0)),
        ],
        out_specs=pl.BlockSpec((TILE, D), lambda i, ntu, te: (i, 0)),
    )
    return pl.pallas_call(
        _ffn_body,
        grid_spec=grid_spec,
        out_shape=jax.ShapeDtypeStruct((PN, D), jnp.float32),
    )(ntu, te, xg, gate_up_w, down_w)


def _comb_body(hg_ref, wp_ref, x2_ref, o_ref):
    k = pl.program_id(1)
    lane = lax.broadcasted_iota(jnp.int32, (SB, 128), 1)
    wk = jnp.sum(jnp.where(lane == k, wp_ref[...], 0.0), axis=1, keepdims=True)
    contrib = hg_ref[...] * wk

    @pl.when(k == 0)
    def _():
        o_ref[...] = x2_ref[...] + contrib

    @pl.when(k > 0)
    def _():
        o_ref[...] = o_ref[...] + contrib


def _combine(hg, wpad, x2):
    return pl.pallas_call(
        _comb_body,
        grid=(NSB, TOPK),
        in_specs=[pl.BlockSpec((SB, D), lambda i, k: (k * NSB + i, 0)),
                  pl.BlockSpec((SB, 128), lambda i, k: (i, 0)),
                  pl.BlockSpec((SB, D), lambda i, k: (i, 0))],
        out_specs=pl.BlockSpec((SB, D), lambda i, k: (i, 0)),
        out_shape=jax.ShapeDtypeStruct((S, D), jnp.float32),
    )(hg, wpad, x2)


_NW = 32


def _sc_dispatch(x3, idx4):
    mesh = plsc.VectorSubcoreMesh(core_axis_name="c", subcore_axis_name="s")
    tok_per_w = S // _NW
    tchunk = 16
    nchunk = tok_per_w // tchunk

    @functools.partial(
        pl.kernel, mesh=mesh,
        out_type=jax.ShapeDtypeStruct((PN, D), jnp.float32),
        scratch_types=[pltpu.VMEM((nchunk, TOPK, tchunk), jnp.int32),
                       pltpu.VMEM((tchunk, D), jnp.float32),
                       pltpu.VMEM((tchunk, D), jnp.float32),
                       pltpu.SemaphoreType.DMA,
                       pltpu.SemaphoreType.DMA],
    )
    def k(x3_hbm, idx_hbm, out_hbm, idx_v, buf0, buf1, rsem, ssem):
        wid = lax.axis_index("s") * 2 + lax.axis_index("c")
        tok_base = wid * tok_per_w
        bufs = (buf0, buf1)
        pltpu.sync_copy(idx_hbm.at[wid], idx_v)
        pltpu.async_copy(x3_hbm.at[pl.ds(tok_base, tchunk)], buf0, rsem)
        for c in range(nchunk):
            buf = bufs[c % 2]
            pltpu.make_async_copy(x3_hbm.at[pl.ds(0, tchunk)], buf,
                                  rsem).wait()
            if c + 1 < nchunk:
                pltpu.async_copy(
                    x3_hbm.at[pl.ds(tok_base + (c + 1) * tchunk, tchunk)],
                    bufs[(c + 1) % 2], rsem)
            for kk in range(TOPK):
                pltpu.async_copy(buf, out_hbm.at[idx_v.at[c, kk]], ssem)
            for kk in range(TOPK):
                pltpu.make_async_copy(x3_hbm.at[pl.ds(0, tchunk)], buf,
                                      ssem).wait()
    return k(x3, idx4)


def _sc_gather_rows(table, idx, nrows, clamp_max):
    mesh = plsc.VectorSubcoreMesh(core_axis_name="c", subcore_axis_name="s")
    rows_per_w = nrows // _NW
    chunk = 24 if rows_per_w % 24 == 0 else 16
    nchunk = rows_per_w // chunk
    dt = table.dtype
    ncol = table.shape[1]

    @functools.partial(
        pl.kernel, mesh=mesh,
        out_type=jax.ShapeDtypeStruct((nrows, ncol), dt),
        scratch_types=[pltpu.VMEM((rows_per_w,), jnp.int32),
                       pltpu.VMEM((chunk, ncol), dt),
                       pltpu.VMEM((chunk, ncol), dt),
                       pltpu.SemaphoreType.DMA,
                       pltpu.SemaphoreType.DMA,
                       pltpu.SemaphoreType.DMA,
                       pltpu.SemaphoreType.DMA],
    )
    def k(tab_hbm, idx_hbm, out_hbm, idx_v, buf0, buf1, gs0, gs1, ws0, ws1):
        wid = lax.axis_index("s") * 2 + lax.axis_index("c")
        base = wid * rows_per_w
        bufs = (buf0, buf1)
        gsems = (gs0, gs1)
        wsems = (ws0, ws1)

        pltpu.sync_copy(idx_hbm.at[pl.ds(base, rows_per_w)], idx_v)
        for h in range(rows_per_w // 16):
            v = idx_v[pl.ds(h * 16, 16)]
            idx_v[pl.ds(h * 16, 16)] = jnp.minimum(jnp.maximum(v, 0), clamp_max)

        def gstart(c, b):
            pltpu.async_copy(tab_hbm.at[idx_v.at[pl.ds(c * chunk, chunk)]],
                             bufs[b], gsems[b])

        def gwait(b):
            pltpu.make_async_copy(tab_hbm.at[pl.ds(0, chunk)], bufs[b],
                                  gsems[b]).wait()

        def wstart(c, b):
            pltpu.async_copy(bufs[b], out_hbm.at[pl.ds(base + c * chunk, chunk)],
                             wsems[b])

        def wwait(b):
            pltpu.make_async_copy(tab_hbm.at[pl.ds(0, chunk)], bufs[b],
                                  wsems[b]).wait()

        gstart(0, 0)

        def step(c, carry):
            even = c % 2 == 0

            @pl.when(c >= 1)
            def _():
                @pl.when(even)
                def _():
                    wwait(1)

                @pl.when(jnp.logical_not(even))
                def _():
                    wwait(0)

            @pl.when(c + 1 < nchunk)
            def _():
                @pl.when(even)
                def _():
                    gstart(c + 1, 1)

                @pl.when(jnp.logical_not(even))
                def _():
                    gstart(c + 1, 0)

            @pl.when(even)
            def _():
                gwait(0)
                wstart(c, 0)

            @pl.when(jnp.logical_not(even))
            def _():
                gwait(1)
                wstart(c, 1)

            return carry

        lax.fori_loop(0, nchunk, step, 0)
        wwait((nchunk - 1) % 2)

    return k(table, idx)


def kernel(x, ln1_w, q_w, k_w, v_w, qn_w, kn_w, o_w, ln2_w, gate_w, gate_up_w, down_w):
    x2d = x.reshape(S, D)

    xn = _rmsnorm(x2d, ln1_w)
    wqkv = jnp.concatenate([q_w, k_w, v_w], axis=0).astype(jnp.bfloat16)
    qkv = _qkv_proj(xn, wqkv)
    q3 = qkv[:, :D].reshape(S, H, HD)
    k3 = qkv[:, D:2 * D].reshape(S, H, HD)
    v3 = qkv[:, 2 * D:].reshape(S, H, HD)

    qr = _rope(q3, qn_w.reshape(1, H, HD))
    kr = _rope(k3, kn_w.reshape(1, H, HD))

    qh = qr.transpose(1, 0, 2).astype(jnp.bfloat16)
    kh = kr.transpose(1, 0, 2).astype(jnp.bfloat16)
    vh = v3.transpose(1, 0, 2).astype(jnp.bfloat16)
    ctxh = _attention(qh, kh, vh)
    ctx = ctxh.transpose(1, 0, 2).reshape(S, D)

    x2 = _oproj_res(ctx, o_w.astype(jnp.bfloat16), x2d)
    x3, wpad, ipad = _router(x2, ln2_w, gate_w)

    rankpad, cnt = _meta(ipad)
    poff, tef = _poff(cnt)
    destpad = _dest(ipad, rankpad, poff)

    te = tef[0, :].astype(jnp.int32)
    dest_i = destpad[:, :TOPK].astype(jnp.int32)
    idx4 = dest_i.reshape(_NW, 4, 16, TOPK).transpose(0, 1, 3, 2)

    ntu = tef[1, 0:1].astype(jnp.int32)
    xg = _sc_dispatch(x3, idx4)
    h_sorted = _ffn(ntu, te, xg, gate_up_w, down_w)

    cidx = dest_i.T.reshape(NA)
    hg = _sc_gather_rows(h_sorted, cidx, NA, PN - 1)

    out = _combine(hg, wpad, x2)
    return out.reshape(1, S, D)
